# Initial kernel scaffold; baseline (speedup 1.0000x reference)
import jax
import jax.numpy as jnp
from jax.experimental import pallas as pl


def kernel(x, edge_index, W1, att_src1, att_dst1, b1, W2, att_src2, att_dst2, b2):
    raise NotImplementedError("write your pallas kernel here")


# speedup vs baseline: 33.5056x; 33.5056x over previous
import functools

import jax
import jax.numpy as jnp
from jax import lax
from jax.experimental import pallas as pl
from jax.experimental.pallas import tpu as pltpu
from jax.experimental.pallas import tpu_sc as plsc

N = 10000
E = 320000
F_IN = 128
HEADS = 8
CH = 8
D1 = HEADS * CH
NCLS = 7

WA = 16
WT2 = 16
W2R = 8

NCORE = 2
NSUB = 16
NW = NCORE * NSUB
LANES = 16

K = 128
NCHUNK = E // K
ROWS_PT = 624
TAIL = N - NSUB * ROWS_PT

RB = 1000
GRID = N // RB


def _tc1_body(x_ref, w1_ref, wta_ref, wad_ref, tabh_ref, taba_ref, tabd_ref):
    x = x_ref[...]
    tabh_ref[...] = jnp.dot(x, w1_ref[...], preferred_element_type=jnp.float32)
    taba_ref[...] = jnp.dot(x, wta_ref[...], preferred_element_type=jnp.float32)
    tabd_ref[...] = jnp.dot(x, wad_ref[...], preferred_element_type=jnp.float32)


def _tc2_body(pn0_ref, pn1_ref, pd0_ref, pd1_ref, tabh_ref, taba_ref,
              tabd_ref, b1_ref, w2_ref, as2_ref, ad2_ref, b64_ref,
              tab2_ref, tabd2_ref):
    h = tabh_ref[...]
    asrc = taba_ref[:, :HEADS]
    a = asrc + tabd_ref[:, :HEADS]
    w = jnp.exp(jnp.maximum(a, 0.2 * a))
    b64 = b64_ref[...]
    num = (pn0_ref[...] + pn1_ref[...]
           + h * jnp.dot(w, b64, preferred_element_type=jnp.float32))
    den = pd0_ref[:, :HEADS] + pd1_ref[:, :HEADS] + w
    den64 = jnp.dot(den, b64, preferred_element_type=jnp.float32)
    h1 = jnp.maximum(num / (den64 + 1e-16) + b1_ref[...], 0.0)
    h2 = jnp.dot(h1, w2_ref[...], preferred_element_type=jnp.float32)
    tab2_ref[:, :NCLS] = h2
    tab2_ref[:, NCLS:NCLS + 1] = jnp.ones((RB, 1), jnp.float32)
    tab2_ref[:, NCLS + 1:NCLS + 2] = jnp.dot(
        h2, as2_ref[...], preferred_element_type=jnp.float32)
    tab2_ref[:, NCLS + 2:WT2] = jnp.zeros((RB, WT2 - NCLS - 2), jnp.float32)
    tabd2_ref[:, :1] = jnp.dot(h2, ad2_ref[...],
                               preferred_element_type=jnp.float32)
    tabd2_ref[:, 1:W2R] = jnp.zeros((RB, W2R - 1), jnp.float32)


def _tc3_body(p0_ref, p1_ref, tab2_ref, tabd2_ref, b2_ref, out_ref):
    h2 = tab2_ref[:, :NCLS]
    a = tab2_ref[:, NCLS + 1:NCLS + 2] + tabd2_ref[:, :1]
    w = jnp.exp(jnp.maximum(a, 0.2 * a))
    s = p0_ref[...] + p1_ref[...]
    num = s[:, :NCLS] + h2 * w
    den = s[:, NCLS:NCLS + 1] + w
    o = num / (den + 1e-16) + b2_ref[...]
    m = jnp.max(o, axis=1, keepdims=True)
    out_ref[...] = o - m - jnp.log(jnp.sum(jnp.exp(o - m), axis=1,
                                           keepdims=True))


def _row_spec(width):
    return pl.BlockSpec((RB, width), lambda i: (i, 0))


def _full_spec(shape):
    return pl.BlockSpec(shape, lambda i: tuple(0 for _ in shape))


def _tc1(x, w1, wta, wad):
    return pl.pallas_call(
        _tc1_body,
        grid=(GRID,),
        in_specs=[_row_spec(F_IN), _full_spec((F_IN, D1)),
                  _full_spec((F_IN, WA)), _full_spec((F_IN, WA))],
        out_specs=(_row_spec(D1), _row_spec(WA), _row_spec(WA)),
        out_shape=(jax.ShapeDtypeStruct((N, D1), jnp.float32),
                   jax.ShapeDtypeStruct((N, WA), jnp.float32),
                   jax.ShapeDtypeStruct((N, WA), jnp.float32)),
    )(x, w1, wta, wad)


def _tc2(pn0, pn1, pd0, pd1, tabh, taba, tabd, b1, w2, as2, ad2, b64):
    return pl.pallas_call(
        _tc2_body,
        grid=(GRID,),
        in_specs=[_row_spec(D1), _row_spec(D1), _row_spec(WA), _row_spec(WA),
                  _row_spec(D1), _row_spec(WA), _row_spec(WA),
                  _full_spec((1, D1)), _full_spec((D1, NCLS)),
                  _full_spec((NCLS, 1)), _full_spec((NCLS, 1)),
                  _full_spec((HEADS, D1))],
        out_specs=(_row_spec(WT2), _row_spec(W2R)),
        out_shape=(jax.ShapeDtypeStruct((N, WT2), jnp.float32),
                   jax.ShapeDtypeStruct((N, W2R), jnp.float32)),
    )(pn0, pn1, pd0, pd1, tabh, taba, tabd, b1, w2, as2, ad2, b64)


def _tc3(p0, p1, tab2, tabd2, b2):
    return pl.pallas_call(
        _tc3_body,
        grid=(GRID,),
        in_specs=[_row_spec(WT2), _row_spec(WT2), _row_spec(WT2),
                  _row_spec(W2R), _full_spec((1, NCLS))],
        out_specs=_row_spec(NCLS),
        out_shape=jax.ShapeDtypeStruct((N, NCLS), jnp.float32),
    )(p0, p1, tab2, tabd2, b2)


_MESH = plsc.VectorSubcoreMesh(core_axis_name="c", subcore_axis_name="s",
                               num_cores=NCORE, num_subcores=NSUB)
_SC_PARAMS = pltpu.CompilerParams(needs_layout_passes=False,
                                  use_tc_tiling_on_sc=False)


def _zero_buf(buf, nrows, width):
    def zrow(i, _):
        def zcol(j, _):
            buf[i, pl.ds(j * LANES, LANES)] = jnp.zeros((LANES,), jnp.float32)
            return 0
        lax.fori_loop(0, width // LANES, zcol, 0)
        return 0
    lax.fori_loop(0, nrows, zrow, 0)


def _over_my_rows(sid, fn):
    base = sid * ROWS_PT
    nfull = ROWS_PT // K
    for t in range(nfull):
        fn(base + K * t, K)
    rem = ROWS_PT - nfull * K
    if rem:
        fn(base + nfull * K, rem)


def _iota16():
    return lax.iota(jnp.int32, LANES)


def _ntask(wid):
    return (NCHUNK - wid + NW - 1) // NW


def _sc1_body(tabh_hbm, taba_hbm, tabd_hbm, src_hbm, dst_hbm, outn_hbm,
              outd_hbm, accn, accd, rows_h, rows_a, rows_d, src_i, dst_i):
    cid = lax.axis_index("c")
    sid = lax.axis_index("s")
    wid = cid * NSUB + sid
    iota = _iota16()

    _zero_buf(rows_h, K, D1)
    _zero_buf(rows_a, K, WA)
    _over_my_rows(sid, lambda b, n: pltpu.sync_copy(
        rows_h.at[pl.ds(0, n)], accn.at[pl.ds(b, n)]))
    _over_my_rows(sid, lambda b, n: pltpu.sync_copy(
        rows_a.at[pl.ds(0, n)], accd.at[pl.ds(b, n)]))

    @pl.when(sid == 0)
    def _zero_tail():
        pltpu.sync_copy(rows_h.at[pl.ds(0, TAIL)],
                        accn.at[pl.ds(NSUB * ROWS_PT, TAIL)])
        pltpu.sync_copy(rows_a.at[pl.ds(0, TAIL)],
                        accd.at[pl.ds(NSUB * ROWS_PT, TAIL)])

    plsc.subcore_barrier()

    def chunk(t, _):
        eb = (wid + NW * t) * K
        pltpu.sync_copy(src_hbm.at[pl.ds(eb, K)], src_i)
        pltpu.sync_copy(dst_hbm.at[pl.ds(eb, K)], dst_i)
        pltpu.sync_copy(tabh_hbm.at[src_i], rows_h)
        pltpu.sync_copy(taba_hbm.at[src_i], rows_a)
        pltpu.sync_copy(tabd_hbm.at[dst_i], rows_d)

        def group(g, _):
            row16 = g * LANES + iota
            for hh in range(HEADS):
                ci = jnp.full((LANES,), hh, jnp.int32)
                adv = plsc.load_gather(rows_d, [row16, ci])
                asv = plsc.load_gather(rows_a, [row16, ci])
                a = asv + adv
                w = jnp.exp(jnp.maximum(a, 0.2 * a))
                plsc.store_scatter(rows_a, [row16, ci], w)
                for cc in range(CH):
                    col = jnp.full((LANES,), hh * CH + cc, jnp.int32)
                    v = plsc.load_gather(rows_h, [row16, col])
                    plsc.store_scatter(rows_h, [row16, col], v * w)
            return 0

        lax.fori_loop(0, K // LANES, group, 0)
        pltpu.sync_copy(rows_h, accn.at[dst_i], add=True)
        pltpu.sync_copy(rows_a, accd.at[dst_i], add=True)
        return 0

    lax.fori_loop(0, _ntask(wid), chunk, 0)
    plsc.subcore_barrier()

    _over_my_rows(sid, lambda b, n: pltpu.sync_copy(
        accn.at[pl.ds(b, n)], outn_hbm.at[cid, pl.ds(b, n)]))
    _over_my_rows(sid, lambda b, n: pltpu.sync_copy(
        accd.at[pl.ds(b, n)], outd_hbm.at[cid, pl.ds(b, n)]))

    @pl.when(sid == 0)
    def _dump_tail():
        pltpu.sync_copy(accn.at[pl.ds(NSUB * ROWS_PT, TAIL)],
                        outn_hbm.at[cid, pl.ds(NSUB * ROWS_PT, TAIL)])
        pltpu.sync_copy(accd.at[pl.ds(NSUB * ROWS_PT, TAIL)],
                        outd_hbm.at[cid, pl.ds(NSUB * ROWS_PT, TAIL)])


def _sc2_body(tab2_hbm, tabd2_hbm, src_hbm, dst_hbm, out_hbm,
              acc, rows, rows_d, src_i, dst_i):
    cid = lax.axis_index("c")
    sid = lax.axis_index("s")
    wid = cid * NSUB + sid
    iota = _iota16()

    _zero_buf(rows, K, WT2)
    _over_my_rows(sid, lambda b, n: pltpu.sync_copy(
        rows.at[pl.ds(0, n)], acc.at[pl.ds(b, n)]))

    @pl.when(sid == 0)
    def _zero_tail():
        pltpu.sync_copy(rows.at[pl.ds(0, TAIL)],
                        acc.at[pl.ds(NSUB * ROWS_PT, TAIL)])

    plsc.subcore_barrier()

    def chunk(t, _):
        eb = (wid + NW * t) * K
        pltpu.sync_copy(src_hbm.at[pl.ds(eb, K)], src_i)
        pltpu.sync_copy(dst_hbm.at[pl.ds(eb, K)], dst_i)
        pltpu.sync_copy(tab2_hbm.at[src_i], rows)
        pltpu.sync_copy(tabd2_hbm.at[dst_i], rows_d)

        def group(g, _):
            row16 = g * LANES + iota
            c0 = jnp.full((LANES,), 0, jnp.int32)
            asv = plsc.load_gather(rows, [row16,
                                          jnp.full((LANES,), NCLS + 1,
                                                   jnp.int32)])
            adv = plsc.load_gather(rows_d, [row16, c0])
            a = asv + adv
            w = jnp.exp(jnp.maximum(a, 0.2 * a))
            for cc in range(W2R):
                col = jnp.full((LANES,), cc, jnp.int32)
                v = plsc.load_gather(rows, [row16, col])
                plsc.store_scatter(rows, [row16, col], v * w)
            return 0

        lax.fori_loop(0, K // LANES, group, 0)
        pltpu.sync_copy(rows, acc.at[dst_i], add=True)
        return 0

    lax.fori_loop(0, _ntask(wid), chunk, 0)
    plsc.subcore_barrier()

    _over_my_rows(sid, lambda b, n: pltpu.sync_copy(
        acc.at[pl.ds(b, n)], out_hbm.at[cid, pl.ds(b, n)]))

    @pl.when(sid == 0)
    def _dump_tail():
        pltpu.sync_copy(acc.at[pl.ds(NSUB * ROWS_PT, TAIL)],
                        out_hbm.at[cid, pl.ds(NSUB * ROWS_PT, TAIL)])


_sc1 = functools.partial(
    pl.kernel,
    out_type=(jax.ShapeDtypeStruct((NCORE, N, D1), jnp.float32),
              jax.ShapeDtypeStruct((NCORE, N, WA), jnp.float32)),
    mesh=_MESH,
    compiler_params=_SC_PARAMS,
    scratch_types=[
        pltpu.VMEM_SHARED((N, D1), jnp.float32),
        pltpu.VMEM_SHARED((N, WA), jnp.float32),
        pltpu.VMEM((K, D1), jnp.float32),
        pltpu.VMEM((K, WA), jnp.float32),
        pltpu.VMEM((K, WA), jnp.float32),
        pltpu.VMEM((K,), jnp.int32),
        pltpu.VMEM((K,), jnp.int32),
    ],
)(_sc1_body)

_sc2 = functools.partial(
    pl.kernel,
    out_type=jax.ShapeDtypeStruct((NCORE, N, WT2), jnp.float32),
    mesh=_MESH,
    compiler_params=_SC_PARAMS,
    scratch_types=[
        pltpu.VMEM_SHARED((N, WT2), jnp.float32),
        pltpu.VMEM((K, WT2), jnp.float32),
        pltpu.VMEM((K, W2R), jnp.float32),
        pltpu.VMEM((K,), jnp.int32),
        pltpu.VMEM((K,), jnp.int32),
    ],
)(_sc2_body)


def kernel(x, edge_index, W1, att_src1, att_dst1, b1, W2, att_src2, att_dst2,
           b2):
    src = edge_index[0]
    dst = edge_index[1]

    eye = jnp.eye(HEADS, dtype=jnp.float32)
    As1 = (att_src1[:, :, None] * eye[:, None, :]).reshape(D1, HEADS)
    Ad1 = (att_dst1[:, :, None] * eye[:, None, :]).reshape(D1, HEADS)
    b64 = jnp.repeat(eye, CH, axis=1)
    pad8 = jnp.zeros((F_IN, WA - HEADS), jnp.float32)
    wta = jnp.concatenate([W1 @ As1, pad8], axis=1)
    wad = jnp.concatenate([W1 @ Ad1, pad8], axis=1)

    tabh, taba, tabd = _tc1(x, W1, wta, wad)
    pn, pd = _sc1(tabh, taba, tabd, src, dst)
    tab2, tabd2 = _tc2(pn[0], pn[1], pd[0], pd[1], tabh, taba, tabd,
                       b1.reshape(1, D1), W2, att_src2.T, att_dst2.T, b64)
    p2 = _sc2(tab2, tabd2, src, dst)
    return _tc3(p2[0], p2[1], tab2, tabd2, b2.reshape(1, NCLS))


# speedup vs baseline: 45.9265x; 1.3707x over previous
import functools

import jax
import jax.numpy as jnp
from jax import lax
from jax.experimental import pallas as pl
from jax.experimental.pallas import tpu as pltpu
from jax.experimental.pallas import tpu_sc as plsc

N = 10000
E = 320000
F_IN = 128
HEADS = 8
CH = 8
D1 = HEADS * CH
NCLS = 7

WA = 16
WT2 = 16
W2R = 8

NCORE = 2
NSUB = 16
NW = NCORE * NSUB
LANES = 16

K = 128
NCHUNK = E // K
NT = NCHUNK // 32
ROWS_PT = 624
TAIL = N - NSUB * ROWS_PT

RB = 1000
GRID = N // RB


def _tc1_body(x_ref, w1_ref, wta_ref, wad_ref, tabh_ref, taba_ref, tabd_ref):
    x = x_ref[...]
    tabh_ref[...] = jnp.dot(x, w1_ref[...], preferred_element_type=jnp.float32)
    taba_ref[...] = jnp.dot(x, wta_ref[...], preferred_element_type=jnp.float32)
    tabd_ref[...] = jnp.dot(x, wad_ref[...], preferred_element_type=jnp.float32)


def _tc2_body(pn0_ref, pn1_ref, pd0_ref, pd1_ref, tabh_ref, taba_ref,
              tabd_ref, b1_ref, w2_ref, as2_ref, ad2_ref, b64_ref,
              tab2_ref, tabd2_ref):
    h = tabh_ref[...]
    asrc = taba_ref[:, :HEADS]
    a = asrc + tabd_ref[:, :HEADS]
    w = jnp.exp(jnp.maximum(a, 0.2 * a))
    b64 = b64_ref[...]
    num = (pn0_ref[...] + pn1_ref[...]
           + h * jnp.dot(w, b64, preferred_element_type=jnp.float32))
    den = pd0_ref[:, :HEADS] + pd1_ref[:, :HEADS] + w
    den64 = jnp.dot(den, b64, preferred_element_type=jnp.float32)
    h1 = jnp.maximum(num / (den64 + 1e-16) + b1_ref[...], 0.0)
    h2 = jnp.dot(h1, w2_ref[...], preferred_element_type=jnp.float32)
    tab2_ref[:, :NCLS] = h2
    tab2_ref[:, NCLS:NCLS + 1] = jnp.ones((RB, 1), jnp.float32)
    tab2_ref[:, NCLS + 1:NCLS + 2] = jnp.dot(
        h2, as2_ref[...], preferred_element_type=jnp.float32)
    tab2_ref[:, NCLS + 2:WT2] = jnp.zeros((RB, WT2 - NCLS - 2), jnp.float32)
    tabd2_ref[:, :1] = jnp.dot(h2, ad2_ref[...],
                               preferred_element_type=jnp.float32)
    tabd2_ref[:, 1:W2R] = jnp.zeros((RB, W2R - 1), jnp.float32)


def _tc3_body(p0_ref, p1_ref, tab2_ref, tabd2_ref, b2_ref, out_ref):
    h2 = tab2_ref[:, :NCLS]
    a = tab2_ref[:, NCLS + 1:NCLS + 2] + tabd2_ref[:, :1]
    w = jnp.exp(jnp.maximum(a, 0.2 * a))
    s = p0_ref[...] + p1_ref[...]
    num = s[:, :NCLS] + h2 * w
    den = s[:, NCLS:NCLS + 1] + w
    o = num / (den + 1e-16) + b2_ref[...]
    m = jnp.max(o, axis=1, keepdims=True)
    out_ref[...] = o - m - jnp.log(jnp.sum(jnp.exp(o - m), axis=1,
                                           keepdims=True))


def _row_spec(width):
    return pl.BlockSpec((RB, width), lambda i: (i, 0))


def _full_spec(shape):
    return pl.BlockSpec(shape, lambda i: tuple(0 for _ in shape))


def _tc1(x, w1, wta, wad):
    return pl.pallas_call(
        _tc1_body,
        grid=(GRID,),
        in_specs=[_row_spec(F_IN), _full_spec((F_IN, D1)),
                  _full_spec((F_IN, WA)), _full_spec((F_IN, WA))],
        out_specs=(_row_spec(D1), _row_spec(WA), _row_spec(WA)),
        out_shape=(jax.ShapeDtypeStruct((N, D1), jnp.float32),
                   jax.ShapeDtypeStruct((N, WA), jnp.float32),
                   jax.ShapeDtypeStruct((N, WA), jnp.float32)),
    )(x, w1, wta, wad)


def _tc2(pn0, pn1, pd0, pd1, tabh, taba, tabd, b1, w2, as2, ad2, b64):
    return pl.pallas_call(
        _tc2_body,
        grid=(GRID,),
        in_specs=[_row_spec(D1), _row_spec(D1), _row_spec(WA), _row_spec(WA),
                  _row_spec(D1), _row_spec(WA), _row_spec(WA),
                  _full_spec((1, D1)), _full_spec((D1, NCLS)),
                  _full_spec((NCLS, 1)), _full_spec((NCLS, 1)),
                  _full_spec((HEADS, D1))],
        out_specs=(_row_spec(WT2), _row_spec(W2R)),
        out_shape=(jax.ShapeDtypeStruct((N, WT2), jnp.float32),
                   jax.ShapeDtypeStruct((N, W2R), jnp.float32)),
    )(pn0, pn1, pd0, pd1, tabh, taba, tabd, b1, w2, as2, ad2, b64)


def _tc3(p0, p1, tab2, tabd2, b2):
    return pl.pallas_call(
        _tc3_body,
        grid=(GRID,),
        in_specs=[_row_spec(WT2), _row_spec(WT2), _row_spec(WT2),
                  _row_spec(W2R), _full_spec((1, NCLS))],
        out_specs=_row_spec(NCLS),
        out_shape=jax.ShapeDtypeStruct((N, NCLS), jnp.float32),
    )(p0, p1, tab2, tabd2, b2)


_MESH = plsc.VectorSubcoreMesh(core_axis_name="c", subcore_axis_name="s",
                               num_cores=NCORE, num_subcores=NSUB)
_SC_PARAMS = pltpu.CompilerParams(needs_layout_passes=False,
                                  use_tc_tiling_on_sc=False)


def _zero_buf(buf, nrows, width):
    def zrow(i, _):
        def zcol(j, _):
            buf[i, pl.ds(j * LANES, LANES)] = jnp.zeros((LANES,), jnp.float32)
            return 0
        lax.fori_loop(0, width // LANES, zcol, 0)
        return 0
    lax.fori_loop(0, nrows, zrow, 0)


def _over_my_rows(sid, fn):
    base = sid * ROWS_PT
    nfull = ROWS_PT // K
    for t in range(nfull):
        fn(base + K * t, K)
    rem = ROWS_PT - nfull * K
    if rem:
        fn(base + nfull * K, rem)


def _iota16():
    return lax.iota(jnp.int32, LANES)


def _edge_pipeline(wid, src2d_hbm, dst2d_hbm, src_all, dst_all, gathers,
                   scatters, compute, gsems, ssems):
    cb = wid * NT

    def fire_g(t, b):
        for tab, bufs, by_dst in gathers:
            idx = (dst_all if by_dst else src_all).at[t]
            pltpu.async_copy(tab.at[idx], bufs[b], gsems[b])

    def wait_g(t, b):
        for tab, bufs, by_dst in gathers:
            idx = (dst_all if by_dst else src_all).at[t]
            pltpu.make_async_copy(tab.at[idx], bufs[b], gsems[b]).wait()

    def fire_s(t, b):
        for acc, bufs in scatters:
            pltpu.async_copy(bufs[b], acc.at[dst_all.at[t]], ssems[b],
                             add=True)

    def wait_s(t, b):
        for acc, bufs in scatters:
            pltpu.make_async_copy(bufs[b], acc.at[dst_all.at[t]],
                                  ssems[b]).wait()

    pltpu.sync_copy(src2d_hbm.at[pl.ds(cb, NT)], src_all.at[pl.ds(0, NT)])
    pltpu.sync_copy(dst2d_hbm.at[pl.ds(cb, NT)], dst_all.at[pl.ds(0, NT)])

    @pl.when(wid < NCHUNK - NW * NT)
    def _extra_idx():
        pltpu.sync_copy(src2d_hbm.at[pl.ds(NW * NT + wid, 1)],
                        src_all.at[pl.ds(NT, 1)])
        pltpu.sync_copy(dst2d_hbm.at[pl.ds(NW * NT + wid, 1)],
                        dst_all.at[pl.ds(NT, 1)])

    fire_g(0, 0)

    def pair(t2, _):
        for b in (0, 1):
            t = 2 * t2 + b
            wait_g(t, b)

            @pl.when(t >= 1)
            def _ws():
                wait_s(t - 1, 1 - b)

            @pl.when(t <= NT - 2)
            def _fg():
                fire_g(t + 1, 1 - b)

            compute(b)
            fire_s(t, b)
        return 0

    lax.fori_loop(0, NT // 2, pair, 0)
    wait_s(NT - 1, 1)

    @pl.when(wid < NCHUNK - NW * NT)
    def _extra():
        for tab, bufs, by_dst in gathers:
            idx = (dst_all if by_dst else src_all).at[NT]
            pltpu.sync_copy(tab.at[idx], bufs[0])
        compute(0)
        for acc, bufs in scatters:
            pltpu.sync_copy(bufs[0], acc.at[dst_all.at[NT]], add=True)


def _sc1_body(tabh_hbm, taba_hbm, tabd_hbm, src2d_hbm, dst2d_hbm, outn_hbm,
              outd_hbm, accn, accd, rh0, rh1, ra0, ra1, rd0, rd1, src_all,
              dst_all, gs0, gs1, ss0, ss1):
    cid = lax.axis_index("c")
    sid = lax.axis_index("s")
    wid = cid * NSUB + sid
    iota = _iota16()
    rows_h = (rh0, rh1)
    rows_a = (ra0, ra1)
    rows_d = (rd0, rd1)

    _zero_buf(rh0, K, D1)
    _zero_buf(ra0, K, WA)
    _over_my_rows(sid, lambda b, n: pltpu.sync_copy(
        rh0.at[pl.ds(0, n)], accn.at[pl.ds(b, n)]))
    _over_my_rows(sid, lambda b, n: pltpu.sync_copy(
        ra0.at[pl.ds(0, n)], accd.at[pl.ds(b, n)]))

    @pl.when(sid == 0)
    def _zero_tail():
        pltpu.sync_copy(rh0.at[pl.ds(0, TAIL)],
                        accn.at[pl.ds(NSUB * ROWS_PT, TAIL)])
        pltpu.sync_copy(ra0.at[pl.ds(0, TAIL)],
                        accd.at[pl.ds(NSUB * ROWS_PT, TAIL)])

    plsc.subcore_barrier()

    def compute(b):
        rh, ra, rd = rows_h[b], rows_a[b], rows_d[b]

        def group(g, _):
            row16 = g * LANES + iota
            for hh in range(HEADS):
                ci = jnp.full((LANES,), hh, jnp.int32)
                adv = plsc.load_gather(rd, [row16, ci])
                asv = plsc.load_gather(ra, [row16, ci])
                a = asv + adv
                w = jnp.exp(jnp.maximum(a, 0.2 * a))
                plsc.store_scatter(ra, [row16, ci], w)
                for cc in range(CH):
                    col = jnp.full((LANES,), hh * CH + cc, jnp.int32)
                    v = plsc.load_gather(rh, [row16, col])
                    plsc.store_scatter(rh, [row16, col], v * w)
            return 0

        lax.fori_loop(0, K // LANES, group, 0)

    _edge_pipeline(
        wid, src2d_hbm, dst2d_hbm, src_all, dst_all,
        gathers=[(tabh_hbm, rows_h, False), (taba_hbm, rows_a, False),
                 (tabd_hbm, rows_d, True)],
        scatters=[(accn, rows_h), (accd, rows_a)],
        compute=compute, gsems=(gs0, gs1), ssems=(ss0, ss1))
    plsc.subcore_barrier()

    _over_my_rows(sid, lambda b, n: pltpu.sync_copy(
        accn.at[pl.ds(b, n)], outn_hbm.at[cid, pl.ds(b, n)]))
    _over_my_rows(sid, lambda b, n: pltpu.sync_copy(
        accd.at[pl.ds(b, n)], outd_hbm.at[cid, pl.ds(b, n)]))

    @pl.when(sid == 0)
    def _dump_tail():
        pltpu.sync_copy(accn.at[pl.ds(NSUB * ROWS_PT, TAIL)],
                        outn_hbm.at[cid, pl.ds(NSUB * ROWS_PT, TAIL)])
        pltpu.sync_copy(accd.at[pl.ds(NSUB * ROWS_PT, TAIL)],
                        outd_hbm.at[cid, pl.ds(NSUB * ROWS_PT, TAIL)])


def _sc2_body(tab2_hbm, tabd2_hbm, src2d_hbm, dst2d_hbm, out_hbm,
              acc, r0, r1, rd0, rd1, src_all, dst_all, gs0, gs1, ss0, ss1):
    cid = lax.axis_index("c")
    sid = lax.axis_index("s")
    wid = cid * NSUB + sid
    iota = _iota16()
    rows = (r0, r1)
    rows_d = (rd0, rd1)

    _zero_buf(r0, K, WT2)
    _over_my_rows(sid, lambda b, n: pltpu.sync_copy(
        r0.at[pl.ds(0, n)], acc.at[pl.ds(b, n)]))

    @pl.when(sid == 0)
    def _zero_tail():
        pltpu.sync_copy(r0.at[pl.ds(0, TAIL)],
                        acc.at[pl.ds(NSUB * ROWS_PT, TAIL)])

    plsc.subcore_barrier()

    def compute(b):
        rw, rd = rows[b], rows_d[b]

        def group(g, _):
            row16 = g * LANES + iota
            asv = plsc.load_gather(rw, [row16,
                                        jnp.full((LANES,), NCLS + 1,
                                                 jnp.int32)])
            adv = plsc.load_gather(rd, [row16,
                                        jnp.full((LANES,), 0, jnp.int32)])
            a = asv + adv
            w = jnp.exp(jnp.maximum(a, 0.2 * a))
            for cc in range(W2R):
                col = jnp.full((LANES,), cc, jnp.int32)
                v = plsc.load_gather(rw, [row16, col])
                plsc.store_scatter(rw, [row16, col], v * w)
            return 0

        lax.fori_loop(0, K // LANES, group, 0)

    _edge_pipeline(
        wid, src2d_hbm, dst2d_hbm, src_all, dst_all,
        gathers=[(tab2_hbm, rows, False), (tabd2_hbm, rows_d, True)],
        scatters=[(acc, rows)],
        compute=compute, gsems=(gs0, gs1), ssems=(ss0, ss1))
    plsc.subcore_barrier()

    _over_my_rows(sid, lambda b, n: pltpu.sync_copy(
        acc.at[pl.ds(b, n)], out_hbm.at[cid, pl.ds(b, n)]))

    @pl.when(sid == 0)
    def _dump_tail():
        pltpu.sync_copy(acc.at[pl.ds(NSUB * ROWS_PT, TAIL)],
                        out_hbm.at[cid, pl.ds(NSUB * ROWS_PT, TAIL)])


_sc1 = functools.partial(
    pl.kernel,
    out_type=(jax.ShapeDtypeStruct((NCORE, N, D1), jnp.float32),
              jax.ShapeDtypeStruct((NCORE, N, WA), jnp.float32)),
    mesh=_MESH,
    compiler_params=_SC_PARAMS,
    scratch_types=[
        pltpu.VMEM_SHARED((N, D1), jnp.float32),
        pltpu.VMEM_SHARED((N, WA), jnp.float32),
        pltpu.VMEM((K, D1), jnp.float32),
        pltpu.VMEM((K, D1), jnp.float32),
        pltpu.VMEM((K, WA), jnp.float32),
        pltpu.VMEM((K, WA), jnp.float32),
        pltpu.VMEM((K, WA), jnp.float32),
        pltpu.VMEM((K, WA), jnp.float32),
        pltpu.VMEM((NT + 1, K), jnp.int32),
        pltpu.VMEM((NT + 1, K), jnp.int32),
        pltpu.SemaphoreType.DMA,
        pltpu.SemaphoreType.DMA,
        pltpu.SemaphoreType.DMA,
        pltpu.SemaphoreType.DMA,
    ],
)(_sc1_body)

_sc2 = functools.partial(
    pl.kernel,
    out_type=jax.ShapeDtypeStruct((NCORE, N, WT2), jnp.float32),
    mesh=_MESH,
    compiler_params=_SC_PARAMS,
    scratch_types=[
        pltpu.VMEM_SHARED((N, WT2), jnp.float32),
        pltpu.VMEM((K, WT2), jnp.float32),
        pltpu.VMEM((K, WT2), jnp.float32),
        pltpu.VMEM((K, W2R), jnp.float32),
        pltpu.VMEM((K, W2R), jnp.float32),
        pltpu.VMEM((NT + 1, K), jnp.int32),
        pltpu.VMEM((NT + 1, K), jnp.int32),
        pltpu.SemaphoreType.DMA,
        pltpu.SemaphoreType.DMA,
        pltpu.SemaphoreType.DMA,
        pltpu.SemaphoreType.DMA,
    ],
)(_sc2_body)


def kernel(x, edge_index, W1, att_src1, att_dst1, b1, W2, att_src2, att_dst2,
           b2):
    src = edge_index[0].reshape(NCHUNK, K)
    dst = edge_index[1].reshape(NCHUNK, K)

    eye = jnp.eye(HEADS, dtype=jnp.float32)
    As1 = (att_src1[:, :, None] * eye[:, None, :]).reshape(D1, HEADS)
    Ad1 = (att_dst1[:, :, None] * eye[:, None, :]).reshape(D1, HEADS)
    b64 = jnp.repeat(eye, CH, axis=1)
    pad8 = jnp.zeros((F_IN, WA - HEADS), jnp.float32)
    wta = jnp.concatenate([W1 @ As1, pad8], axis=1)
    wad = jnp.concatenate([W1 @ Ad1, pad8], axis=1)

    tabh, taba, tabd = _tc1(x, W1, wta, wad)
    pn, pd = _sc1(tabh, taba, tabd, src, dst)
    tab2, tabd2 = _tc2(pn[0], pn[1], pd[0], pd[1], tabh, taba, tabd,
                       b1.reshape(1, D1), W2, att_src2.T, att_dst2.T, b64)
    p2 = _sc2(tab2, tabd2, src, dst)
    return _tc3(p2[0], p2[1], tab2, tabd2, b2.reshape(1, NCLS))


# speedup vs baseline: 47.7433x; 1.0396x over previous
import functools

import jax
import jax.numpy as jnp
from jax import lax
from jax.experimental import pallas as pl
from jax.experimental.pallas import tpu as pltpu
from jax.experimental.pallas import tpu_sc as plsc

N = 10000
E = 320000
F_IN = 128
HEADS = 8
CH = 8
D1 = HEADS * CH
NCLS = 7

WA = 16
WT2 = 16
W2R = 8

NCORE = 2
NSUB = 16
NW = NCORE * NSUB
LANES = 16

K = 128
NCHUNK = E // K
NT = NCHUNK // 32
ROWS_PT = 624
TAIL = N - NSUB * ROWS_PT

RB = 1000
GRID = N // RB


def _tc1_body(x_ref, w1_ref, wta_ref, wad_ref, tabh_ref, taba_ref, tabd_ref):
    x = x_ref[...]
    tabh_ref[...] = jnp.dot(x, w1_ref[...], preferred_element_type=jnp.float32)
    taba_ref[...] = jnp.dot(x, wta_ref[...], preferred_element_type=jnp.float32)
    tabd_ref[...] = jnp.dot(x, wad_ref[...], preferred_element_type=jnp.float32)


def _tc2_body(pn0_ref, pn1_ref, pd0_ref, pd1_ref, tabh_ref, taba_ref,
              tabd_ref, b1_ref, w2_ref, as2_ref, ad2_ref, b64_ref,
              tab2_ref, tabd2_ref):
    h = tabh_ref[...]
    asrc = taba_ref[:, :HEADS]
    a = asrc + tabd_ref[:, :HEADS]
    w = jnp.exp(jnp.maximum(a, 0.2 * a))
    b64 = b64_ref[...]
    num = (pn0_ref[...] + pn1_ref[...]
           + h * jnp.dot(w, b64, preferred_element_type=jnp.float32))
    den = pd0_ref[:, :HEADS] + pd1_ref[:, :HEADS] + w
    den64 = jnp.dot(den, b64, preferred_element_type=jnp.float32)
    h1 = jnp.maximum(num / (den64 + 1e-16) + b1_ref[...], 0.0)
    h2 = jnp.dot(h1, w2_ref[...], preferred_element_type=jnp.float32)
    tab2_ref[:, :NCLS] = h2
    tab2_ref[:, NCLS:NCLS + 1] = jnp.ones((RB, 1), jnp.float32)
    tab2_ref[:, NCLS + 1:NCLS + 2] = jnp.dot(
        h2, as2_ref[...], preferred_element_type=jnp.float32)
    tab2_ref[:, NCLS + 2:WT2] = jnp.zeros((RB, WT2 - NCLS - 2), jnp.float32)
    tabd2_ref[:, :1] = jnp.dot(h2, ad2_ref[...],
                               preferred_element_type=jnp.float32)
    tabd2_ref[:, 1:W2R] = jnp.zeros((RB, W2R - 1), jnp.float32)


def _tc3_body(p0_ref, p1_ref, tab2_ref, tabd2_ref, b2_ref, out_ref):
    h2 = tab2_ref[:, :NCLS]
    a = tab2_ref[:, NCLS + 1:NCLS + 2] + tabd2_ref[:, :1]
    w = jnp.exp(jnp.maximum(a, 0.2 * a))
    s = p0_ref[...] + p1_ref[...]
    num = s[:, :NCLS] + h2 * w
    den = s[:, NCLS:NCLS + 1] + w
    o = num / (den + 1e-16) + b2_ref[...]
    m = jnp.max(o, axis=1, keepdims=True)
    out_ref[...] = o - m - jnp.log(jnp.sum(jnp.exp(o - m), axis=1,
                                           keepdims=True))


def _row_spec(width):
    return pl.BlockSpec((RB, width), lambda i: (i, 0))


def _full_spec(shape):
    return pl.BlockSpec(shape, lambda i: tuple(0 for _ in shape))


def _tc1(x, w1, wta, wad):
    return pl.pallas_call(
        _tc1_body,
        grid=(GRID,),
        in_specs=[_row_spec(F_IN), _full_spec((F_IN, D1)),
                  _full_spec((F_IN, WA)), _full_spec((F_IN, WA))],
        out_specs=(_row_spec(D1), _row_spec(WA), _row_spec(WA)),
        out_shape=(jax.ShapeDtypeStruct((N, D1), jnp.float32),
                   jax.ShapeDtypeStruct((N, WA), jnp.float32),
                   jax.ShapeDtypeStruct((N, WA), jnp.float32)),
    )(x, w1, wta, wad)


def _tc2(pn0, pn1, pd0, pd1, tabh, taba, tabd, b1, w2, as2, ad2, b64):
    return pl.pallas_call(
        _tc2_body,
        grid=(GRID,),
        in_specs=[_row_spec(D1), _row_spec(D1), _row_spec(WA), _row_spec(WA),
                  _row_spec(D1), _row_spec(WA), _row_spec(WA),
                  _full_spec((1, D1)), _full_spec((D1, NCLS)),
                  _full_spec((NCLS, 1)), _full_spec((NCLS, 1)),
                  _full_spec((HEADS, D1))],
        out_specs=(_row_spec(WT2), _row_spec(W2R)),
        out_shape=(jax.ShapeDtypeStruct((N, WT2), jnp.float32),
                   jax.ShapeDtypeStruct((N, W2R), jnp.float32)),
    )(pn0, pn1, pd0, pd1, tabh, taba, tabd, b1, w2, as2, ad2, b64)


def _tc3(p0, p1, tab2, tabd2, b2):
    return pl.pallas_call(
        _tc3_body,
        grid=(GRID,),
        in_specs=[_row_spec(WT2), _row_spec(WT2), _row_spec(WT2),
                  _row_spec(W2R), _full_spec((1, NCLS))],
        out_specs=_row_spec(NCLS),
        out_shape=jax.ShapeDtypeStruct((N, NCLS), jnp.float32),
    )(p0, p1, tab2, tabd2, b2)


_MESH = plsc.VectorSubcoreMesh(core_axis_name="c", subcore_axis_name="s",
                               num_cores=NCORE, num_subcores=NSUB)
_SC_PARAMS = pltpu.CompilerParams(needs_layout_passes=False,
                                  use_tc_tiling_on_sc=False)


def _zero_buf(buf, nrows, width):
    def zrow(i, _):
        def zcol(j, _):
            buf[i, pl.ds(j * LANES, LANES)] = jnp.zeros((LANES,), jnp.float32)
            return 0
        lax.fori_loop(0, width // LANES, zcol, 0)
        return 0
    lax.fori_loop(0, nrows, zrow, 0)


def _over_my_rows(sid, fn):
    base = sid * ROWS_PT
    nfull = ROWS_PT // K
    for t in range(nfull):
        fn(base + K * t, K)
    rem = ROWS_PT - nfull * K
    if rem:
        fn(base + nfull * K, rem)


def _iota16():
    return lax.iota(jnp.int32, LANES)


def _edge_pipeline(wid, src2d_hbm, dst2d_hbm, src_all, dst_all, gathers,
                   scatters, compute, gsems, ssems):
    cb = wid * NT

    def fire_g(t, b):
        for tab, bufs, by_dst in gathers:
            idx = (dst_all if by_dst else src_all).at[t]
            pltpu.async_copy(tab.at[idx], bufs[b], gsems[b])

    def wait_g(t, b):
        for tab, bufs, by_dst in gathers:
            idx = (dst_all if by_dst else src_all).at[t]
            pltpu.make_async_copy(tab.at[idx], bufs[b], gsems[b]).wait()

    def fire_s(t, b):
        for acc, bufs in scatters:
            pltpu.async_copy(bufs[b], acc.at[dst_all.at[t]], ssems[b],
                             add=True)

    def wait_s(t, b):
        for acc, bufs in scatters:
            pltpu.make_async_copy(bufs[b], acc.at[dst_all.at[t]],
                                  ssems[b]).wait()

    pltpu.sync_copy(src2d_hbm.at[pl.ds(cb, NT)], src_all.at[pl.ds(0, NT)])
    pltpu.sync_copy(dst2d_hbm.at[pl.ds(cb, NT)], dst_all.at[pl.ds(0, NT)])

    @pl.when(wid < NCHUNK - NW * NT)
    def _extra_idx():
        pltpu.sync_copy(src2d_hbm.at[pl.ds(NW * NT + wid, 1)],
                        src_all.at[pl.ds(NT, 1)])
        pltpu.sync_copy(dst2d_hbm.at[pl.ds(NW * NT + wid, 1)],
                        dst_all.at[pl.ds(NT, 1)])

    fire_g(0, 0)

    def pair(t2, _):
        for b in (0, 1):
            t = 2 * t2 + b
            wait_g(t, b)

            @pl.when(t >= 1)
            def _ws():
                wait_s(t - 1, 1 - b)

            @pl.when(t <= NT - 2)
            def _fg():
                fire_g(t + 1, 1 - b)

            compute(b)
            fire_s(t, b)
        return 0

    lax.fori_loop(0, NT // 2, pair, 0)
    wait_s(NT - 1, 1)

    @pl.when(wid < NCHUNK - NW * NT)
    def _extra():
        for tab, bufs, by_dst in gathers:
            idx = (dst_all if by_dst else src_all).at[NT]
            pltpu.sync_copy(tab.at[idx], bufs[0])
        compute(0)
        for acc, bufs in scatters:
            pltpu.sync_copy(bufs[0], acc.at[dst_all.at[NT]], add=True)


def _sc1_body(tabh_hbm, taba_hbm, tabd_hbm, src2d_hbm, dst2d_hbm, outn_hbm,
              outd_hbm, accn, accd, rh0, rh1, ra0, ra1, rd0, rd1, src_all,
              dst_all, gs0, gs1, ss0, ss1):
    cid = lax.axis_index("c")
    sid = lax.axis_index("s")
    wid = cid * NSUB + sid
    iota = _iota16()
    rows_h = (rh0, rh1)
    rows_a = (ra0, ra1)
    rows_d = (rd0, rd1)

    _zero_buf(rh0, K, D1)
    _zero_buf(ra0, K, WA)
    _over_my_rows(sid, lambda b, n: pltpu.sync_copy(
        rh0.at[pl.ds(0, n)], accn.at[pl.ds(b, n)]))
    _over_my_rows(sid, lambda b, n: pltpu.sync_copy(
        ra0.at[pl.ds(0, n)], accd.at[pl.ds(b, n)]))

    @pl.when(sid == 0)
    def _zero_tail():
        pltpu.sync_copy(rh0.at[pl.ds(0, TAIL)],
                        accn.at[pl.ds(NSUB * ROWS_PT, TAIL)])
        pltpu.sync_copy(ra0.at[pl.ds(0, TAIL)],
                        accd.at[pl.ds(NSUB * ROWS_PT, TAIL)])

    plsc.subcore_barrier()

    def compute(b):
        rh, ra, rd = rows_h[b], rows_a[b], rows_d[b]

        def group(g2, _):
            r16a = (2 * g2) * LANES + iota
            r16b = r16a + LANES
            for hh in range(HEADS):
                ci = jnp.full((LANES,), hh, jnp.int32)
                ava = plsc.load_gather(rd, [r16a, ci])
                avb = plsc.load_gather(rd, [r16b, ci])
                asa = plsc.load_gather(ra, [r16a, ci])
                asb = plsc.load_gather(ra, [r16b, ci])
                aa = asa + ava
                ab = asb + avb
                wa = jnp.exp(jnp.maximum(aa, 0.2 * aa))
                wb = jnp.exp(jnp.maximum(ab, 0.2 * ab))
                plsc.store_scatter(ra, [r16a, ci], wa)
                plsc.store_scatter(ra, [r16b, ci], wb)
                for cc in range(CH):
                    col = jnp.full((LANES,), hh * CH + cc, jnp.int32)
                    va = plsc.load_gather(rh, [r16a, col])
                    vb = plsc.load_gather(rh, [r16b, col])
                    plsc.store_scatter(rh, [r16a, col], va * wa)
                    plsc.store_scatter(rh, [r16b, col], vb * wb)
            return 0

        lax.fori_loop(0, K // LANES // 2, group, 0)

    _edge_pipeline(
        wid, src2d_hbm, dst2d_hbm, src_all, dst_all,
        gathers=[(tabh_hbm, rows_h, False), (taba_hbm, rows_a, False),
                 (tabd_hbm, rows_d, True)],
        scatters=[(accn, rows_h), (accd, rows_a)],
        compute=compute, gsems=(gs0, gs1), ssems=(ss0, ss1))
    plsc.subcore_barrier()

    _over_my_rows(sid, lambda b, n: pltpu.sync_copy(
        accn.at[pl.ds(b, n)], outn_hbm.at[cid, pl.ds(b, n)]))
    _over_my_rows(sid, lambda b, n: pltpu.sync_copy(
        accd.at[pl.ds(b, n)], outd_hbm.at[cid, pl.ds(b, n)]))

    @pl.when(sid == 0)
    def _dump_tail():
        pltpu.sync_copy(accn.at[pl.ds(NSUB * ROWS_PT, TAIL)],
                        outn_hbm.at[cid, pl.ds(NSUB * ROWS_PT, TAIL)])
        pltpu.sync_copy(accd.at[pl.ds(NSUB * ROWS_PT, TAIL)],
                        outd_hbm.at[cid, pl.ds(NSUB * ROWS_PT, TAIL)])


def _sc2_body(tab2_hbm, tabd2_hbm, src2d_hbm, dst2d_hbm, out_hbm,
              acc, r0, r1, rd0, rd1, src_all, dst_all, gs0, gs1, ss0, ss1):
    cid = lax.axis_index("c")
    sid = lax.axis_index("s")
    wid = cid * NSUB + sid
    iota = _iota16()
    rows = (r0, r1)
    rows_d = (rd0, rd1)

    _zero_buf(r0, K, WT2)
    _over_my_rows(sid, lambda b, n: pltpu.sync_copy(
        r0.at[pl.ds(0, n)], acc.at[pl.ds(b, n)]))

    @pl.when(sid == 0)
    def _zero_tail():
        pltpu.sync_copy(r0.at[pl.ds(0, TAIL)],
                        acc.at[pl.ds(NSUB * ROWS_PT, TAIL)])

    plsc.subcore_barrier()

    def compute(b):
        rw, rd = rows[b], rows_d[b]

        def group(g2, _):
            r16a = (2 * g2) * LANES + iota
            r16b = r16a + LANES
            cs = jnp.full((LANES,), NCLS + 1, jnp.int32)
            c0 = jnp.full((LANES,), 0, jnp.int32)
            asa = plsc.load_gather(rw, [r16a, cs])
            asb = plsc.load_gather(rw, [r16b, cs])
            ava = plsc.load_gather(rd, [r16a, c0])
            avb = plsc.load_gather(rd, [r16b, c0])
            aa = asa + ava
            ab = asb + avb
            wa = jnp.exp(jnp.maximum(aa, 0.2 * aa))
            wb = jnp.exp(jnp.maximum(ab, 0.2 * ab))
            for cc in range(W2R):
                col = jnp.full((LANES,), cc, jnp.int32)
                va = plsc.load_gather(rw, [r16a, col])
                vb = plsc.load_gather(rw, [r16b, col])
                plsc.store_scatter(rw, [r16a, col], va * wa)
                plsc.store_scatter(rw, [r16b, col], vb * wb)
            return 0

        lax.fori_loop(0, K // LANES // 2, group, 0)

    _edge_pipeline(
        wid, src2d_hbm, dst2d_hbm, src_all, dst_all,
        gathers=[(tab2_hbm, rows, False), (tabd2_hbm, rows_d, True)],
        scatters=[(acc, rows)],
        compute=compute, gsems=(gs0, gs1), ssems=(ss0, ss1))
    plsc.subcore_barrier()

    _over_my_rows(sid, lambda b, n: pltpu.sync_copy(
        acc.at[pl.ds(b, n)], out_hbm.at[cid, pl.ds(b, n)]))

    @pl.when(sid == 0)
    def _dump_tail():
        pltpu.sync_copy(acc.at[pl.ds(NSUB * ROWS_PT, TAIL)],
                        out_hbm.at[cid, pl.ds(NSUB * ROWS_PT, TAIL)])


_sc1 = functools.partial(
    pl.kernel,
    out_type=(jax.ShapeDtypeStruct((NCORE, N, D1), jnp.float32),
              jax.ShapeDtypeStruct((NCORE, N, WA), jnp.float32)),
    mesh=_MESH,
    compiler_params=_SC_PARAMS,
    scratch_types=[
        pltpu.VMEM_SHARED((N, D1), jnp.float32),
        pltpu.VMEM_SHARED((N, WA), jnp.float32),
        pltpu.VMEM((K, D1), jnp.float32),
        pltpu.VMEM((K, D1), jnp.float32),
        pltpu.VMEM((K, WA), jnp.float32),
        pltpu.VMEM((K, WA), jnp.float32),
        pltpu.VMEM((K, WA), jnp.float32),
        pltpu.VMEM((K, WA), jnp.float32),
        pltpu.VMEM((NT + 1, K), jnp.int32),
        pltpu.VMEM((NT + 1, K), jnp.int32),
        pltpu.SemaphoreType.DMA,
        pltpu.SemaphoreType.DMA,
        pltpu.SemaphoreType.DMA,
        pltpu.SemaphoreType.DMA,
    ],
)(_sc1_body)

_sc2 = functools.partial(
    pl.kernel,
    out_type=jax.ShapeDtypeStruct((NCORE, N, WT2), jnp.float32),
    mesh=_MESH,
    compiler_params=_SC_PARAMS,
    scratch_types=[
        pltpu.VMEM_SHARED((N, WT2), jnp.float32),
        pltpu.VMEM((K, WT2), jnp.float32),
        pltpu.VMEM((K, WT2), jnp.float32),
        pltpu.VMEM((K, W2R), jnp.float32),
        pltpu.VMEM((K, W2R), jnp.float32),
        pltpu.VMEM((NT + 1, K), jnp.int32),
        pltpu.VMEM((NT + 1, K), jnp.int32),
        pltpu.SemaphoreType.DMA,
        pltpu.SemaphoreType.DMA,
        pltpu.SemaphoreType.DMA,
        pltpu.SemaphoreType.DMA,
    ],
)(_sc2_body)


def kernel(x, edge_index, W1, att_src1, att_dst1, b1, W2, att_src2, att_dst2,
           b2):
    src = edge_index[0].reshape(NCHUNK, K)
    dst = edge_index[1].reshape(NCHUNK, K)

    eye = jnp.eye(HEADS, dtype=jnp.float32)
    As1 = (att_src1[:, :, None] * eye[:, None, :]).reshape(D1, HEADS)
    Ad1 = (att_dst1[:, :, None] * eye[:, None, :]).reshape(D1, HEADS)
    b64 = jnp.repeat(eye, CH, axis=1)
    pad8 = jnp.zeros((F_IN, WA - HEADS), jnp.float32)
    wta = jnp.concatenate([W1 @ As1, pad8], axis=1)
    wad = jnp.concatenate([W1 @ Ad1, pad8], axis=1)

    tabh, taba, tabd = _tc1(x, W1, wta, wad)
    pn, pd = _sc1(tabh, taba, tabd, src, dst)
    tab2, tabd2 = _tc2(pn[0], pn[1], pd[0], pd[1], tabh, taba, tabd,
                       b1.reshape(1, D1), W2, att_src2.T, att_dst2.T, b64)
    p2 = _sc2(tab2, tabd2, src, dst)
    return _tc3(p2[0], p2[1], tab2, tabd2, b2.reshape(1, NCLS))


# speedup vs baseline: 73.0559x; 1.5302x over previous
import functools

import jax
import jax.numpy as jnp
from jax import lax
from jax.experimental import pallas as pl
from jax.experimental.pallas import tpu as pltpu
from jax.experimental.pallas import tpu_sc as plsc

N = 10000
E = 320000
F_IN = 128
HEADS = 8
CH = 8
D1 = HEADS * CH
NCLS = 7

ROW1 = 80
ACOL = D1
WT2 = 16

NCORE = 2
NSUB = 16
NW = NCORE * NSUB
LANES = 16

K = 128
NCHUNK = E // K
NT = NCHUNK // 32
ROWS_PT = 624
TAIL = N - NSUB * ROWS_PT

AD1_R = N * HEADS // 128
AD2_R = 80

RB = 1000
GRID = N // RB


WA = 16


def _tc1_body(x_ref, wtab_ref, wad_ref, tab_ref, adst_ref):
    x = x_ref[...]
    tab_ref[...] = jnp.dot(x, wtab_ref[...], preferred_element_type=jnp.float32)
    adst_ref[...] = jnp.dot(x, wad_ref[...], preferred_element_type=jnp.float32)


def _tc2_body(p0_ref, p1_ref, tab_ref, adst_ref, b1_ref, w2_ref, as2_ref,
              ad2_ref, b64_ref, tab2_ref, adst2_ref):
    tab = tab_ref[...]
    h = tab[:, :D1]
    asrc = tab[:, ACOL:ACOL + HEADS]
    a = asrc + adst_ref[:, :HEADS]
    w = jnp.exp(jnp.maximum(a, 0.2 * a))
    b64 = b64_ref[...]
    s = p0_ref[...] + p1_ref[...]
    num = s[:, :D1] + h * jnp.dot(w, b64, preferred_element_type=jnp.float32)
    den = s[:, ACOL:ACOL + HEADS] + w
    den64 = jnp.dot(den, b64, preferred_element_type=jnp.float32)
    h1 = jnp.maximum(num / (den64 + 1e-16) + b1_ref[...], 0.0)
    h2 = jnp.dot(h1, w2_ref[...], preferred_element_type=jnp.float32)
    tab2_ref[:, :NCLS] = h2
    tab2_ref[:, NCLS:NCLS + 1] = jnp.ones((RB, 1), jnp.float32)
    tab2_ref[:, NCLS + 1:NCLS + 2] = jnp.dot(
        h2, as2_ref[...], preferred_element_type=jnp.float32)
    tab2_ref[:, NCLS + 2:WT2] = jnp.zeros((RB, WT2 - NCLS - 2), jnp.float32)
    adst2_ref[:, :1] = jnp.dot(h2, ad2_ref[...],
                               preferred_element_type=jnp.float32)
    adst2_ref[:, 1:HEADS] = jnp.zeros((RB, HEADS - 1), jnp.float32)


def _tc3_body(p0_ref, p1_ref, tab2_ref, adst2_ref, b2_ref, out_ref):
    h2 = tab2_ref[:, :NCLS]
    a = tab2_ref[:, NCLS + 1:NCLS + 2] + adst2_ref[:, :1]
    w = jnp.exp(jnp.maximum(a, 0.2 * a))
    s = p0_ref[...] + p1_ref[...]
    num = s[:, :NCLS] + h2 * w
    den = s[:, NCLS:NCLS + 1] + w
    o = num / (den + 1e-16) + b2_ref[...]
    m = jnp.max(o, axis=1, keepdims=True)
    out_ref[...] = o - m - jnp.log(jnp.sum(jnp.exp(o - m), axis=1,
                                           keepdims=True))


def _row_spec(width):
    return pl.BlockSpec((RB, width), lambda i: (i, 0))


def _full_spec(shape):
    return pl.BlockSpec(shape, lambda i: tuple(0 for _ in shape))


def _tc1(x, wtab, wad):
    return pl.pallas_call(
        _tc1_body,
        grid=(GRID,),
        in_specs=[_row_spec(F_IN), _full_spec((F_IN, ROW1)),
                  _full_spec((F_IN, WA))],
        out_specs=(_row_spec(ROW1), _row_spec(WA)),
        out_shape=(jax.ShapeDtypeStruct((N, ROW1), jnp.float32),
                   jax.ShapeDtypeStruct((N, WA), jnp.float32)),
    )(x, wtab, wad)


def _tc2(p0, p1, tab, adst, b1, w2, as2, ad2, b64):
    return pl.pallas_call(
        _tc2_body,
        grid=(GRID,),
        in_specs=[_row_spec(ROW1), _row_spec(ROW1), _row_spec(ROW1),
                  _row_spec(WA), _full_spec((1, D1)),
                  _full_spec((D1, NCLS)), _full_spec((NCLS, 1)),
                  _full_spec((NCLS, 1)), _full_spec((HEADS, D1))],
        out_specs=(_row_spec(WT2), _row_spec(HEADS)),
        out_shape=(jax.ShapeDtypeStruct((N, WT2), jnp.float32),
                   jax.ShapeDtypeStruct((N, HEADS), jnp.float32)),
    )(p0, p1, tab, adst, b1, w2, as2, ad2, b64)


def _tc3(p0, p1, tab2, adst2, b2):
    return pl.pallas_call(
        _tc3_body,
        grid=(GRID,),
        in_specs=[_row_spec(WT2), _row_spec(WT2), _row_spec(WT2),
                  _row_spec(HEADS), _full_spec((1, NCLS))],
        out_specs=_row_spec(NCLS),
        out_shape=jax.ShapeDtypeStruct((N, NCLS), jnp.float32),
    )(p0, p1, tab2, adst2, b2)


_MESH = plsc.VectorSubcoreMesh(core_axis_name="c", subcore_axis_name="s",
                               num_cores=NCORE, num_subcores=NSUB)
_SC_PARAMS = pltpu.CompilerParams(needs_layout_passes=False,
                                  use_tc_tiling_on_sc=False)


def _zero_buf(buf, nrows, width):
    def zrow(i, _):
        def zcol(j, _):
            buf[i, pl.ds(j * LANES, LANES)] = jnp.zeros((LANES,), jnp.float32)
            return 0
        lax.fori_loop(0, width // LANES, zcol, 0)
        return 0
    lax.fori_loop(0, nrows, zrow, 0)


def _over_my_rows(sid, fn):
    base = sid * ROWS_PT
    nfull = ROWS_PT // K
    for t in range(nfull):
        fn(base + K * t, K)
    rem = ROWS_PT - nfull * K
    if rem:
        fn(base + nfull * K, rem)


def _iota16():
    return lax.iota(jnp.int32, LANES)


def _edge_pipeline(wid, src2d_hbm, dst2d_hbm, src_all, dst_all, gathers,
                   scatters, compute, gsems, ssems):
    cb = wid * NT

    def fire_g(t, b):
        for tab, bufs, by_dst in gathers:
            idx = (dst_all if by_dst else src_all).at[t]
            pltpu.async_copy(tab.at[idx], bufs[b], gsems[b])

    def wait_g(t, b):
        for tab, bufs, by_dst in gathers:
            idx = (dst_all if by_dst else src_all).at[t]
            pltpu.make_async_copy(tab.at[idx], bufs[b], gsems[b]).wait()

    def fire_s(t, b):
        for acc, bufs in scatters:
            pltpu.async_copy(bufs[b], acc.at[dst_all.at[t]], ssems[b],
                             add=True)

    def wait_s(t, b):
        for acc, bufs in scatters:
            pltpu.make_async_copy(bufs[b], acc.at[dst_all.at[t]],
                                  ssems[b]).wait()

    pltpu.sync_copy(src2d_hbm.at[pl.ds(cb, NT)], src_all.at[pl.ds(0, NT)])
    pltpu.sync_copy(dst2d_hbm.at[pl.ds(cb, NT)], dst_all.at[pl.ds(0, NT)])

    @pl.when(wid < NCHUNK - NW * NT)
    def _extra_idx():
        pltpu.sync_copy(src2d_hbm.at[pl.ds(NW * NT + wid, 1)],
                        src_all.at[pl.ds(NT, 1)])
        pltpu.sync_copy(dst2d_hbm.at[pl.ds(NW * NT + wid, 1)],
                        dst_all.at[pl.ds(NT, 1)])

    fire_g(0, 0)

    def pair(t2, _):
        for b in (0, 1):
            t = 2 * t2 + b
            wait_g(t, b)

            @pl.when(t >= 1)
            def _ws():
                wait_s(t - 1, 1 - b)

            @pl.when(t <= NT - 2)
            def _fg():
                fire_g(t + 1, 1 - b)

            compute(t, b)
            fire_s(t, b)
        return 0

    lax.fori_loop(0, NT // 2, pair, 0)
    wait_s(NT - 1, 1)

    @pl.when(wid < NCHUNK - NW * NT)
    def _extra():
        for tab, bufs, by_dst in gathers:
            idx = (dst_all if by_dst else src_all).at[NT]
            pltpu.sync_copy(tab.at[idx], bufs[0])
        compute(NT, 0)
        for acc, bufs in scatters:
            pltpu.sync_copy(bufs[0], acc.at[dst_all.at[NT]], add=True)


def _sc1_body(tab_hbm, tabd_hbm, src2d_hbm, dst2d_hbm, out_hbm,
              acc, r0, r1, rd0, rd1, src_all, dst_all, gs0, gs1, ss0, ss1):
    cid = lax.axis_index("c")
    sid = lax.axis_index("s")
    wid = cid * NSUB + sid
    iota = _iota16()
    rows = (r0, r1)
    rows_d = (rd0, rd1)

    _zero_buf(r0, K, ROW1)
    _over_my_rows(sid, lambda b, n: pltpu.sync_copy(
        r0.at[pl.ds(0, n)], acc.at[pl.ds(b, n)]))

    @pl.when(sid == 0)
    def _zero_tail():
        pltpu.sync_copy(r0.at[pl.ds(0, TAIL)],
                        acc.at[pl.ds(NSUB * ROWS_PT, TAIL)])

    plsc.subcore_barrier()

    def compute(t, b):
        rw = rows[b]
        rd = rows_d[b]

        def group(g2, _):
            r16a = (2 * g2) * LANES + iota
            r16b = r16a + LANES
            for hh in range(HEADS):
                ci = jnp.full((LANES,), ACOL + hh, jnp.int32)
                ch = jnp.full((LANES,), hh, jnp.int32)
                ava = plsc.load_gather(rd, [r16a, ch])
                avb = plsc.load_gather(rd, [r16b, ch])
                asa = plsc.load_gather(rw, [r16a, ci])
                asb = plsc.load_gather(rw, [r16b, ci])
                aa = asa + ava
                ab = asb + avb
                wa = jnp.exp(jnp.maximum(aa, 0.2 * aa))
                wb = jnp.exp(jnp.maximum(ab, 0.2 * ab))
                plsc.store_scatter(rw, [r16a, ci], wa)
                plsc.store_scatter(rw, [r16b, ci], wb)
                for cc in range(CH):
                    col = jnp.full((LANES,), hh * CH + cc, jnp.int32)
                    va = plsc.load_gather(rw, [r16a, col])
                    vb = plsc.load_gather(rw, [r16b, col])
                    plsc.store_scatter(rw, [r16a, col], va * wa)
                    plsc.store_scatter(rw, [r16b, col], vb * wb)
            return 0

        lax.fori_loop(0, K // LANES // 2, group, 0)

    _edge_pipeline(
        wid, src2d_hbm, dst2d_hbm, src_all, dst_all,
        gathers=[(tab_hbm, rows, False), (tabd_hbm, rows_d, True)],
        scatters=[(acc, rows)],
        compute=compute, gsems=(gs0, gs1), ssems=(ss0, ss1))
    plsc.subcore_barrier()

    _over_my_rows(sid, lambda b, n: pltpu.sync_copy(
        acc.at[pl.ds(b, n)], out_hbm.at[cid, pl.ds(b, n)]))

    @pl.when(sid == 0)
    def _dump_tail():
        pltpu.sync_copy(acc.at[pl.ds(NSUB * ROWS_PT, TAIL)],
                        out_hbm.at[cid, pl.ds(NSUB * ROWS_PT, TAIL)])


def _sc2_body(tab2_hbm, tabd2_hbm, src2d_hbm, dst2d_hbm, out_hbm,
              acc, r0, r1, rd0, rd1, src_all, dst_all, gs0, gs1, ss0, ss1):
    cid = lax.axis_index("c")
    sid = lax.axis_index("s")
    wid = cid * NSUB + sid
    iota = _iota16()
    rows = (r0, r1)
    rows_d = (rd0, rd1)

    _zero_buf(r0, K, WT2)
    _over_my_rows(sid, lambda b, n: pltpu.sync_copy(
        r0.at[pl.ds(0, n)], acc.at[pl.ds(b, n)]))

    @pl.when(sid == 0)
    def _zero_tail():
        pltpu.sync_copy(r0.at[pl.ds(0, TAIL)],
                        acc.at[pl.ds(NSUB * ROWS_PT, TAIL)])

    plsc.subcore_barrier()

    def compute(t, b):
        rw = rows[b]
        rd = rows_d[b]

        def group(g2, _):
            r16a = (2 * g2) * LANES + iota
            r16b = r16a + LANES
            cs = jnp.full((LANES,), NCLS + 1, jnp.int32)
            c0 = jnp.full((LANES,), 0, jnp.int32)
            ava = plsc.load_gather(rd, [r16a, c0])
            avb = plsc.load_gather(rd, [r16b, c0])
            asa = plsc.load_gather(rw, [r16a, cs])
            asb = plsc.load_gather(rw, [r16b, cs])
            aa = asa + ava
            ab = asb + avb
            wa = jnp.exp(jnp.maximum(aa, 0.2 * aa))
            wb = jnp.exp(jnp.maximum(ab, 0.2 * ab))
            for cc in range(CH):
                col = jnp.full((LANES,), cc, jnp.int32)
                va = plsc.load_gather(rw, [r16a, col])
                vb = plsc.load_gather(rw, [r16b, col])
                plsc.store_scatter(rw, [r16a, col], va * wa)
                plsc.store_scatter(rw, [r16b, col], vb * wb)
            return 0

        lax.fori_loop(0, K // LANES // 2, group, 0)

    _edge_pipeline(
        wid, src2d_hbm, dst2d_hbm, src_all, dst_all,
        gathers=[(tab2_hbm, rows, False), (tabd2_hbm, rows_d, True)],
        scatters=[(acc, rows)],
        compute=compute, gsems=(gs0, gs1), ssems=(ss0, ss1))
    plsc.subcore_barrier()

    _over_my_rows(sid, lambda b, n: pltpu.sync_copy(
        acc.at[pl.ds(b, n)], out_hbm.at[cid, pl.ds(b, n)]))

    @pl.when(sid == 0)
    def _dump_tail():
        pltpu.sync_copy(acc.at[pl.ds(NSUB * ROWS_PT, TAIL)],
                        out_hbm.at[cid, pl.ds(NSUB * ROWS_PT, TAIL)])


_sc1 = functools.partial(
    pl.kernel,
    out_type=jax.ShapeDtypeStruct((NCORE, N, ROW1), jnp.float32),
    mesh=_MESH,
    compiler_params=_SC_PARAMS,
    scratch_types=[
        pltpu.VMEM_SHARED((N, ROW1), jnp.float32),
        pltpu.VMEM((K, ROW1), jnp.float32),
        pltpu.VMEM((K, ROW1), jnp.float32),
        pltpu.VMEM((K, WA), jnp.float32),
        pltpu.VMEM((K, WA), jnp.float32),
        pltpu.VMEM((NT + 1, K), jnp.int32),
        pltpu.VMEM((NT + 1, K), jnp.int32),
        pltpu.SemaphoreType.DMA,
        pltpu.SemaphoreType.DMA,
        pltpu.SemaphoreType.DMA,
        pltpu.SemaphoreType.DMA,
    ],
)(_sc1_body)

_sc2 = functools.partial(
    pl.kernel,
    out_type=jax.ShapeDtypeStruct((NCORE, N, WT2), jnp.float32),
    mesh=_MESH,
    compiler_params=_SC_PARAMS,
    scratch_types=[
        pltpu.VMEM_SHARED((N, WT2), jnp.float32),
        pltpu.VMEM((K, WT2), jnp.float32),
        pltpu.VMEM((K, WT2), jnp.float32),
        pltpu.VMEM((K, HEADS), jnp.float32),
        pltpu.VMEM((K, HEADS), jnp.float32),
        pltpu.VMEM((NT + 1, K), jnp.int32),
        pltpu.VMEM((NT + 1, K), jnp.int32),
        pltpu.SemaphoreType.DMA,
        pltpu.SemaphoreType.DMA,
        pltpu.SemaphoreType.DMA,
        pltpu.SemaphoreType.DMA,
    ],
)(_sc2_body)


def kernel(x, edge_index, W1, att_src1, att_dst1, b1, W2, att_src2, att_dst2,
           b2):
    src = edge_index[0].reshape(NCHUNK, K)
    dst = edge_index[1].reshape(NCHUNK, K)

    eye = jnp.eye(HEADS, dtype=jnp.float32)
    As1 = (att_src1[:, :, None] * eye[:, None, :]).reshape(D1, HEADS)
    Ad1 = (att_dst1[:, :, None] * eye[:, None, :]).reshape(D1, HEADS)
    b64 = jnp.repeat(eye, CH, axis=1)
    pad8 = jnp.zeros((F_IN, WA - HEADS), jnp.float32)
    wtab = jnp.concatenate(
        [W1, W1 @ As1, jnp.zeros((F_IN, ROW1 - D1 - HEADS), jnp.float32)],
        axis=1)
    wad = jnp.concatenate([W1 @ Ad1, pad8], axis=1)

    tab1, tabd = _tc1(x, wtab, wad)
    p1 = _sc1(tab1, tabd, src, dst)
    tab2, tabd2 = _tc2(p1[0], p1[1], tab1, tabd, b1.reshape(1, D1),
                       W2, att_src2.T, att_dst2.T, b64)
    p2 = _sc2(tab2, tabd2, src, dst)
    return _tc3(p2[0], p2[1], tab2, tabd2, b2.reshape(1, NCLS))


# speedup vs baseline: 78.3385x; 1.0723x over previous
import functools

import jax
import jax.numpy as jnp
from jax import lax
from jax.experimental import pallas as pl
from jax.experimental.pallas import tpu as pltpu
from jax.experimental.pallas import tpu_sc as plsc

N = 10000
E = 320000
F_IN = 128
HEADS = 8
CH = 8
D1 = HEADS * CH
NCLS = 7

ROW1 = 80
ACOL = D1
WT2 = 16

NCORE = 2
NSUB = 16
NW = NCORE * NSUB
LANES = 16

K = 128
NCHUNK = E // K
NT = NCHUNK // 32
ROWS_PT = 624
TAIL = N - NSUB * ROWS_PT

AD1_R = N * HEADS // 128
AD2_R = 80

RB = 1000
GRID = N // RB


WA = 16


def _tc1_body(x_ref, wtab_ref, wad_ref, tab_ref, adst_ref):
    x = x_ref[...]
    tab_ref[...] = jnp.dot(x, wtab_ref[...], preferred_element_type=jnp.float32)
    adst_ref[...] = jnp.dot(x, wad_ref[...], preferred_element_type=jnp.float32)


def _tc2_body(p0_ref, p1_ref, tab_ref, adst_ref, b1_ref, w2_ref, as2_ref,
              ad2_ref, b64_ref, tab2_ref, adst2_ref):
    tab = tab_ref[...]
    h = tab[:, :D1]
    asrc = tab[:, ACOL:ACOL + HEADS]
    a = asrc + adst_ref[:, :HEADS]
    w = jnp.exp(jnp.maximum(a, 0.2 * a))
    b64 = b64_ref[...]
    s = p0_ref[...] + p1_ref[...]
    num = s[:, :D1] + h * jnp.dot(w, b64, preferred_element_type=jnp.float32)
    den = s[:, ACOL:ACOL + HEADS] + w
    den64 = jnp.dot(den, b64, preferred_element_type=jnp.float32)
    h1 = jnp.maximum(num / (den64 + 1e-16) + b1_ref[...], 0.0)
    h2 = jnp.dot(h1, w2_ref[...], preferred_element_type=jnp.float32)
    tab2_ref[:, :NCLS] = h2
    tab2_ref[:, NCLS:NCLS + 1] = jnp.ones((RB, 1), jnp.float32)
    tab2_ref[:, NCLS + 1:NCLS + 2] = jnp.dot(
        h2, as2_ref[...], preferred_element_type=jnp.float32)
    tab2_ref[:, NCLS + 2:WT2] = jnp.zeros((RB, WT2 - NCLS - 2), jnp.float32)
    adst2_ref[:, :1] = jnp.dot(h2, ad2_ref[...],
                               preferred_element_type=jnp.float32)
    adst2_ref[:, 1:HEADS] = jnp.zeros((RB, HEADS - 1), jnp.float32)


def _tc3_body(p0_ref, p1_ref, tab2_ref, adst2_ref, b2_ref, out_ref):
    h2 = tab2_ref[:, :NCLS]
    a = tab2_ref[:, NCLS + 1:NCLS + 2] + adst2_ref[:, :1]
    w = jnp.exp(jnp.maximum(a, 0.2 * a))
    s = p0_ref[...] + p1_ref[...]
    num = s[:, :NCLS] + h2 * w
    den = s[:, NCLS:NCLS + 1] + w
    o = num / (den + 1e-16) + b2_ref[...]
    m = jnp.max(o, axis=1, keepdims=True)
    out_ref[...] = o - m - jnp.log(jnp.sum(jnp.exp(o - m), axis=1,
                                           keepdims=True))


def _row_spec(width):
    return pl.BlockSpec((RB, width), lambda i: (i, 0))


def _full_spec(shape):
    return pl.BlockSpec(shape, lambda i: tuple(0 for _ in shape))


def _tc1(x, wtab, wad):
    return pl.pallas_call(
        _tc1_body,
        grid=(GRID,),
        in_specs=[_row_spec(F_IN), _full_spec((F_IN, ROW1)),
                  _full_spec((F_IN, WA))],
        out_specs=(_row_spec(ROW1), _row_spec(WA)),
        out_shape=(jax.ShapeDtypeStruct((N, ROW1), jnp.float32),
                   jax.ShapeDtypeStruct((N, WA), jnp.float32)),
    )(x, wtab, wad)


def _tc2(p0, p1, tab, adst, b1, w2, as2, ad2, b64):
    return pl.pallas_call(
        _tc2_body,
        grid=(GRID,),
        in_specs=[_row_spec(ROW1), _row_spec(ROW1), _row_spec(ROW1),
                  _row_spec(WA), _full_spec((1, D1)),
                  _full_spec((D1, NCLS)), _full_spec((NCLS, 1)),
                  _full_spec((NCLS, 1)), _full_spec((HEADS, D1))],
        out_specs=(_row_spec(WT2), _row_spec(HEADS)),
        out_shape=(jax.ShapeDtypeStruct((N, WT2), jnp.float32),
                   jax.ShapeDtypeStruct((N, HEADS), jnp.float32)),
    )(p0, p1, tab, adst, b1, w2, as2, ad2, b64)


def _tc3(p0, p1, tab2, adst2, b2):
    return pl.pallas_call(
        _tc3_body,
        grid=(GRID,),
        in_specs=[_row_spec(WT2), _row_spec(WT2), _row_spec(WT2),
                  _row_spec(HEADS), _full_spec((1, NCLS))],
        out_specs=_row_spec(NCLS),
        out_shape=jax.ShapeDtypeStruct((N, NCLS), jnp.float32),
    )(p0, p1, tab2, adst2, b2)


_MESH = plsc.VectorSubcoreMesh(core_axis_name="c", subcore_axis_name="s",
                               num_cores=NCORE, num_subcores=NSUB)
_SC_PARAMS = pltpu.CompilerParams(needs_layout_passes=False,
                                  use_tc_tiling_on_sc=False)


def _zero_buf(buf, nrows, width):
    def zrow(i, _):
        def zcol(j, _):
            buf[i, pl.ds(j * LANES, LANES)] = jnp.zeros((LANES,), jnp.float32)
            return 0
        lax.fori_loop(0, width // LANES, zcol, 0)
        return 0
    lax.fori_loop(0, nrows, zrow, 0)


def _over_my_rows(sid, fn):
    base = sid * ROWS_PT
    nfull = ROWS_PT // K
    for t in range(nfull):
        fn(base + K * t, K)
    rem = ROWS_PT - nfull * K
    if rem:
        fn(base + nfull * K, rem)


def _iota16():
    return lax.iota(jnp.int32, LANES)


def _edge_pipeline(wid, src2d_hbm, dst2d_hbm, src_all, dst_all, gathers,
                   scatters, compute, gsems, ssems):
    cb = wid * NT

    def fire_g(t, b):
        for tab, bufs, by_dst in gathers:
            idx = (dst_all if by_dst else src_all).at[t]
            pltpu.async_copy(tab.at[idx], bufs[b], gsems[b])

    def wait_g(t, b):
        for tab, bufs, by_dst in gathers:
            idx = (dst_all if by_dst else src_all).at[t]
            pltpu.make_async_copy(tab.at[idx], bufs[b], gsems[b]).wait()

    def fire_s(t, b):
        for acc, bufs in scatters:
            pltpu.async_copy(bufs[b], acc.at[dst_all.at[t]], ssems[b],
                             add=True)

    def wait_s(t, b):
        for acc, bufs in scatters:
            pltpu.make_async_copy(bufs[b], acc.at[dst_all.at[t]],
                                  ssems[b]).wait()

    pltpu.sync_copy(src2d_hbm.at[pl.ds(cb, NT)], src_all.at[pl.ds(0, NT)])
    pltpu.sync_copy(dst2d_hbm.at[pl.ds(cb, NT)], dst_all.at[pl.ds(0, NT)])

    @pl.when(wid < NCHUNK - NW * NT)
    def _extra_idx():
        pltpu.sync_copy(src2d_hbm.at[pl.ds(NW * NT + wid, 1)],
                        src_all.at[pl.ds(NT, 1)])
        pltpu.sync_copy(dst2d_hbm.at[pl.ds(NW * NT + wid, 1)],
                        dst_all.at[pl.ds(NT, 1)])

    fire_g(0, 0)

    def pair(t2, _):
        for b in (0, 1):
            t = 2 * t2 + b
            wait_g(t, b)

            @pl.when(t >= 1)
            def _ws():
                wait_s(t - 1, 1 - b)

            @pl.when(t <= NT - 2)
            def _fg():
                fire_g(t + 1, 1 - b)

            compute(t, b)
            fire_s(t, b)
        return 0

    lax.fori_loop(0, NT // 2, pair, 0)
    wait_s(NT - 1, 1)

    @pl.when(wid < NCHUNK - NW * NT)
    def _extra():
        for tab, bufs, by_dst in gathers:
            idx = (dst_all if by_dst else src_all).at[NT]
            pltpu.sync_copy(tab.at[idx], bufs[0])
        compute(NT, 0)
        for acc, bufs in scatters:
            pltpu.sync_copy(bufs[0], acc.at[dst_all.at[NT]], add=True)


def _sc1_body(tab_hbm, tabd_hbm, src2d_hbm, dst2d_hbm, out_hbm,
              acc, r0, r1, rd0, rd1, src_all, dst_all, gs0, gs1, ss0, ss1):
    cid = lax.axis_index("c")
    sid = lax.axis_index("s")
    wid = cid * NSUB + sid
    iota = _iota16()
    rows = (r0, r1)
    rows_d = (rd0, rd1)

    _zero_buf(r0, K, ROW1)
    _over_my_rows(sid, lambda b, n: pltpu.sync_copy(
        r0.at[pl.ds(0, n)], acc.at[pl.ds(b, n)]))

    @pl.when(sid == 0)
    def _zero_tail():
        pltpu.sync_copy(r0.at[pl.ds(0, TAIL)],
                        acc.at[pl.ds(NSUB * ROWS_PT, TAIL)])

    plsc.subcore_barrier()

    def compute(t, b):
        rw = rows[b]
        rd = rows_d[b]

        def group(g4, _):
            r16s = [(4 * g4 + k) * LANES + iota for k in range(4)]
            for hh in range(HEADS):
                ci = jnp.full((LANES,), ACOL + hh, jnp.int32)
                ch = jnp.full((LANES,), hh, jnp.int32)
                avs = [plsc.load_gather(rd, [r, ch]) for r in r16s]
                ass = [plsc.load_gather(rw, [r, ci]) for r in r16s]
                aas = [s + v for s, v in zip(ass, avs)]
                ws = [jnp.exp(jnp.maximum(a, 0.2 * a)) for a in aas]
                for r, w in zip(r16s, ws):
                    plsc.store_scatter(rw, [r, ci], w)
                for cc in range(CH):
                    col = jnp.full((LANES,), hh * CH + cc, jnp.int32)
                    vs = [plsc.load_gather(rw, [r, col]) for r in r16s]
                    for r, v, w in zip(r16s, vs, ws):
                        plsc.store_scatter(rw, [r, col], v * w)
            return 0

        lax.fori_loop(0, K // LANES // 4, group, 0)

    _edge_pipeline(
        wid, src2d_hbm, dst2d_hbm, src_all, dst_all,
        gathers=[(tab_hbm, rows, False), (tabd_hbm, rows_d, True)],
        scatters=[(acc, rows)],
        compute=compute, gsems=(gs0, gs1), ssems=(ss0, ss1))
    plsc.subcore_barrier()

    _over_my_rows(sid, lambda b, n: pltpu.sync_copy(
        acc.at[pl.ds(b, n)], out_hbm.at[cid, pl.ds(b, n)]))

    @pl.when(sid == 0)
    def _dump_tail():
        pltpu.sync_copy(acc.at[pl.ds(NSUB * ROWS_PT, TAIL)],
                        out_hbm.at[cid, pl.ds(NSUB * ROWS_PT, TAIL)])


def _sc2_body(tab2_hbm, tabd2_hbm, src2d_hbm, dst2d_hbm, out_hbm,
              acc, r0, r1, rd0, rd1, src_all, dst_all, gs0, gs1, ss0, ss1):
    cid = lax.axis_index("c")
    sid = lax.axis_index("s")
    wid = cid * NSUB + sid
    iota = _iota16()
    rows = (r0, r1)
    rows_d = (rd0, rd1)

    _zero_buf(r0, K, WT2)
    _over_my_rows(sid, lambda b, n: pltpu.sync_copy(
        r0.at[pl.ds(0, n)], acc.at[pl.ds(b, n)]))

    @pl.when(sid == 0)
    def _zero_tail():
        pltpu.sync_copy(r0.at[pl.ds(0, TAIL)],
                        acc.at[pl.ds(NSUB * ROWS_PT, TAIL)])

    plsc.subcore_barrier()

    def compute(t, b):
        rw = rows[b]
        rd = rows_d[b]

        def group(g4, _):
            r16s = [(4 * g4 + k) * LANES + iota for k in range(4)]
            cs = jnp.full((LANES,), NCLS + 1, jnp.int32)
            c0 = jnp.full((LANES,), 0, jnp.int32)
            avs = [plsc.load_gather(rd, [r, c0]) for r in r16s]
            ass = [plsc.load_gather(rw, [r, cs]) for r in r16s]
            aas = [s + v for s, v in zip(ass, avs)]
            ws = [jnp.exp(jnp.maximum(a, 0.2 * a)) for a in aas]
            for cc in range(CH):
                col = jnp.full((LANES,), cc, jnp.int32)
                vs = [plsc.load_gather(rw, [r, col]) for r in r16s]
                for r, v, w in zip(r16s, vs, ws):
                    plsc.store_scatter(rw, [r, col], v * w)
            return 0

        lax.fori_loop(0, K // LANES // 4, group, 0)

    _edge_pipeline(
        wid, src2d_hbm, dst2d_hbm, src_all, dst_all,
        gathers=[(tab2_hbm, rows, False), (tabd2_hbm, rows_d, True)],
        scatters=[(acc, rows)],
        compute=compute, gsems=(gs0, gs1), ssems=(ss0, ss1))
    plsc.subcore_barrier()

    _over_my_rows(sid, lambda b, n: pltpu.sync_copy(
        acc.at[pl.ds(b, n)], out_hbm.at[cid, pl.ds(b, n)]))

    @pl.when(sid == 0)
    def _dump_tail():
        pltpu.sync_copy(acc.at[pl.ds(NSUB * ROWS_PT, TAIL)],
                        out_hbm.at[cid, pl.ds(NSUB * ROWS_PT, TAIL)])


_sc1 = functools.partial(
    pl.kernel,
    out_type=jax.ShapeDtypeStruct((NCORE, N, ROW1), jnp.float32),
    mesh=_MESH,
    compiler_params=_SC_PARAMS,
    scratch_types=[
        pltpu.VMEM_SHARED((N, ROW1), jnp.float32),
        pltpu.VMEM((K, ROW1), jnp.float32),
        pltpu.VMEM((K, ROW1), jnp.float32),
        pltpu.VMEM((K, WA), jnp.float32),
        pltpu.VMEM((K, WA), jnp.float32),
        pltpu.VMEM((NT + 1, K), jnp.int32),
        pltpu.VMEM((NT + 1, K), jnp.int32),
        pltpu.SemaphoreType.DMA,
        pltpu.SemaphoreType.DMA,
        pltpu.SemaphoreType.DMA,
        pltpu.SemaphoreType.DMA,
    ],
)(_sc1_body)

_sc2 = functools.partial(
    pl.kernel,
    out_type=jax.ShapeDtypeStruct((NCORE, N, WT2), jnp.float32),
    mesh=_MESH,
    compiler_params=_SC_PARAMS,
    scratch_types=[
        pltpu.VMEM_SHARED((N, WT2), jnp.float32),
        pltpu.VMEM((K, WT2), jnp.float32),
        pltpu.VMEM((K, WT2), jnp.float32),
        pltpu.VMEM((K, HEADS), jnp.float32),
        pltpu.VMEM((K, HEADS), jnp.float32),
        pltpu.VMEM((NT + 1, K), jnp.int32),
        pltpu.VMEM((NT + 1, K), jnp.int32),
        pltpu.SemaphoreType.DMA,
        pltpu.SemaphoreType.DMA,
        pltpu.SemaphoreType.DMA,
        pltpu.SemaphoreType.DMA,
    ],
)(_sc2_body)


def kernel(x, edge_index, W1, att_src1, att_dst1, b1, W2, att_src2, att_dst2,
           b2):
    src = edge_index[0].reshape(NCHUNK, K)
    dst = edge_index[1].reshape(NCHUNK, K)

    eye = jnp.eye(HEADS, dtype=jnp.float32)
    As1 = (att_src1[:, :, None] * eye[:, None, :]).reshape(D1, HEADS)
    Ad1 = (att_dst1[:, :, None] * eye[:, None, :]).reshape(D1, HEADS)
    b64 = jnp.repeat(eye, CH, axis=1)
    pad8 = jnp.zeros((F_IN, WA - HEADS), jnp.float32)
    wtab = jnp.concatenate(
        [W1, W1 @ As1, jnp.zeros((F_IN, ROW1 - D1 - HEADS), jnp.float32)],
        axis=1)
    wad = jnp.concatenate([W1 @ Ad1, pad8], axis=1)

    tab1, tabd = _tc1(x, wtab, wad)
    p1 = _sc1(tab1, tabd, src, dst)
    tab2, tabd2 = _tc2(p1[0], p1[1], tab1, tabd, b1.reshape(1, D1),
                       W2, att_src2.T, att_dst2.T, b64)
    p2 = _sc2(tab2, tabd2, src, dst)
    return _tc3(p2[0], p2[1], tab2, tabd2, b2.reshape(1, NCLS))


# speedup vs baseline: 78.3568x; 1.0002x over previous
import functools

import jax
import jax.numpy as jnp
from jax import lax
from jax.experimental import pallas as pl
from jax.experimental.pallas import tpu as pltpu
from jax.experimental.pallas import tpu_sc as plsc

N = 10000
E = 320000
F_IN = 128
HEADS = 8
CH = 8
D1 = HEADS * CH
NCLS = 7

ROW1 = 80
ACOL = D1
WT2 = 16

NCORE = 2
NSUB = 16
NW = NCORE * NSUB
LANES = 16

K = 128
NCHUNK = E // K
NT = NCHUNK // 32
ROWS_PT = 624
TAIL = N - NSUB * ROWS_PT

RB = 1000
GRID = N // RB


WA = 16


def _tc1_body(x_ref, wtab_ref, wad_ref, tab_ref, adst_ref):
    x = x_ref[...]
    tab_ref[...] = jnp.dot(x, wtab_ref[...], preferred_element_type=jnp.float32)
    adst_ref[...] = jnp.dot(x, wad_ref[...], preferred_element_type=jnp.float32)


def _tc2_body(p0_ref, p1_ref, tab_ref, adst_ref, b1_ref, w2_ref, as2_ref,
              ad2_ref, b64_ref, tab2_ref, adst2_ref):
    tab = tab_ref[...]
    h = tab[:, :D1]
    asrc = tab[:, ACOL:ACOL + HEADS]
    a = asrc + adst_ref[:, :HEADS]
    w = jnp.exp(jnp.maximum(a, 0.2 * a))
    b64 = b64_ref[...]
    s = p0_ref[...] + p1_ref[...]
    num = s[:, :D1] + h * jnp.dot(w, b64, preferred_element_type=jnp.float32)
    den = s[:, ACOL:ACOL + HEADS] + w
    den64 = jnp.dot(den, b64, preferred_element_type=jnp.float32)
    h1 = jnp.maximum(num / (den64 + 1e-16) + b1_ref[...], 0.0)
    h2 = jnp.dot(h1, w2_ref[...], preferred_element_type=jnp.float32)
    tab2_ref[:, :NCLS] = h2
    tab2_ref[:, NCLS:NCLS + 1] = jnp.ones((RB, 1), jnp.float32)
    tab2_ref[:, NCLS + 1:NCLS + 2] = jnp.dot(
        h2, as2_ref[...], preferred_element_type=jnp.float32)
    tab2_ref[:, NCLS + 2:WT2] = jnp.zeros((RB, WT2 - NCLS - 2), jnp.float32)
    adst2_ref[:, :1] = jnp.dot(h2, ad2_ref[...],
                               preferred_element_type=jnp.float32)
    adst2_ref[:, 1:HEADS] = jnp.zeros((RB, HEADS - 1), jnp.float32)


def _tc3_body(p0_ref, p1_ref, tab2_ref, adst2_ref, b2_ref, out_ref):
    h2 = tab2_ref[:, :NCLS]
    a = tab2_ref[:, NCLS + 1:NCLS + 2] + adst2_ref[:, :1]
    w = jnp.exp(jnp.maximum(a, 0.2 * a))
    s = p0_ref[...] + p1_ref[...]
    num = s[:, :NCLS] + h2 * w
    den = s[:, NCLS:NCLS + 1] + w
    o = num / (den + 1e-16) + b2_ref[...]
    m = jnp.max(o, axis=1, keepdims=True)
    out_ref[...] = o - m - jnp.log(jnp.sum(jnp.exp(o - m), axis=1,
                                           keepdims=True))


def _row_spec(width):
    return pl.BlockSpec((RB, width), lambda i: (i, 0))


def _full_spec(shape):
    return pl.BlockSpec(shape, lambda i: tuple(0 for _ in shape))


def _tc1(x, wtab, wad):
    return pl.pallas_call(
        _tc1_body,
        grid=(GRID,),
        in_specs=[_row_spec(F_IN), _full_spec((F_IN, ROW1)),
                  _full_spec((F_IN, WA))],
        out_specs=(_row_spec(ROW1), _row_spec(WA)),
        out_shape=(jax.ShapeDtypeStruct((N, ROW1), jnp.float32),
                   jax.ShapeDtypeStruct((N, WA), jnp.float32)),
    )(x, wtab, wad)


def _tc2(p0, p1, tab, adst, b1, w2, as2, ad2, b64):
    return pl.pallas_call(
        _tc2_body,
        grid=(GRID,),
        in_specs=[_row_spec(ROW1), _row_spec(ROW1), _row_spec(ROW1),
                  _row_spec(WA), _full_spec((1, D1)),
                  _full_spec((D1, NCLS)), _full_spec((NCLS, 1)),
                  _full_spec((NCLS, 1)), _full_spec((HEADS, D1))],
        out_specs=(_row_spec(WT2), _row_spec(HEADS)),
        out_shape=(jax.ShapeDtypeStruct((N, WT2), jnp.float32),
                   jax.ShapeDtypeStruct((N, HEADS), jnp.float32)),
    )(p0, p1, tab, adst, b1, w2, as2, ad2, b64)


def _tc3(p0, p1, tab2, adst2, b2):
    return pl.pallas_call(
        _tc3_body,
        grid=(GRID,),
        in_specs=[_row_spec(WT2), _row_spec(WT2), _row_spec(WT2),
                  _row_spec(HEADS), _full_spec((1, NCLS))],
        out_specs=_row_spec(NCLS),
        out_shape=jax.ShapeDtypeStruct((N, NCLS), jnp.float32),
    )(p0, p1, tab2, adst2, b2)


_MESH = plsc.VectorSubcoreMesh(core_axis_name="c", subcore_axis_name="s",
                               num_cores=NCORE, num_subcores=NSUB)
_SC_PARAMS = pltpu.CompilerParams(needs_layout_passes=False,
                                  use_tc_tiling_on_sc=False)


def _zero_buf(buf, nrows, width):
    def zrow(i, _):
        def zcol(j, _):
            buf[i, pl.ds(j * LANES, LANES)] = jnp.zeros((LANES,), jnp.float32)
            return 0
        lax.fori_loop(0, width // LANES, zcol, 0)
        return 0
    lax.fori_loop(0, nrows, zrow, 0)


def _over_my_rows(sid, fn):
    base = sid * ROWS_PT
    nfull = ROWS_PT // K
    for t in range(nfull):
        fn(base + K * t, K)
    rem = ROWS_PT - nfull * K
    if rem:
        fn(base + nfull * K, rem)


def _iota16():
    return lax.iota(jnp.int32, LANES)


def _edge_pipeline(wid, src2d_hbm, dst2d_hbm, src_all, dst_all, gathers,
                   scatters, compute, gsems, ssems):
    cb = wid * NT

    def fire_g(t, b):
        for tab, bufs, by_dst in gathers:
            idx = (dst_all if by_dst else src_all).at[t]
            pltpu.async_copy(tab.at[idx], bufs[b], gsems[b])

    def wait_g(t, b):
        for tab, bufs, by_dst in gathers:
            idx = (dst_all if by_dst else src_all).at[t]
            pltpu.make_async_copy(tab.at[idx], bufs[b], gsems[b]).wait()

    def fire_s(t, b):
        for acc, bufs in scatters:
            pltpu.async_copy(bufs[b], acc.at[dst_all.at[t]], ssems[b],
                             add=True)

    def wait_s(t, b):
        for acc, bufs in scatters:
            pltpu.make_async_copy(bufs[b], acc.at[dst_all.at[t]],
                                  ssems[b]).wait()

    pltpu.sync_copy(src2d_hbm.at[pl.ds(cb, NT)], src_all.at[pl.ds(0, NT)])
    pltpu.sync_copy(dst2d_hbm.at[pl.ds(cb, NT)], dst_all.at[pl.ds(0, NT)])

    @pl.when(wid < NCHUNK - NW * NT)
    def _extra_idx():
        pltpu.sync_copy(src2d_hbm.at[pl.ds(NW * NT + wid, 1)],
                        src_all.at[pl.ds(NT, 1)])
        pltpu.sync_copy(dst2d_hbm.at[pl.ds(NW * NT + wid, 1)],
                        dst_all.at[pl.ds(NT, 1)])

    fire_g(0, 0)

    def pair(t2, _):
        for b in (0, 1):
            t = 2 * t2 + b
            wait_g(t, b)

            @pl.when(t >= 1)
            def _ws():
                wait_s(t - 1, 1 - b)

            @pl.when(t <= NT - 2)
            def _fg():
                fire_g(t + 1, 1 - b)

            compute(t, b)
            fire_s(t, b)
        return 0

    lax.fori_loop(0, NT // 2, pair, 0)
    wait_s(NT - 1, 1)

    @pl.when(wid < NCHUNK - NW * NT)
    def _extra():
        for tab, bufs, by_dst in gathers:
            idx = (dst_all if by_dst else src_all).at[NT]
            pltpu.sync_copy(tab.at[idx], bufs[0])
        compute(NT, 0)
        for acc, bufs in scatters:
            pltpu.sync_copy(bufs[0], acc.at[dst_all.at[NT]], add=True)


def _sc1_body(tab_hbm, tabd_hbm, src2d_hbm, dst2d_hbm, out_hbm,
              acc, r0, r1, rd0, rd1, src_all, dst_all, gs0, gs1, ss0, ss1):
    cid = lax.axis_index("c")
    sid = lax.axis_index("s")
    wid = cid * NSUB + sid
    iota = _iota16()
    rows = (r0, r1)
    rows_d = (rd0, rd1)

    _zero_buf(r0, K, ROW1)
    _over_my_rows(sid, lambda b, n: pltpu.sync_copy(
        r0.at[pl.ds(0, n)], acc.at[pl.ds(b, n)]))

    @pl.when(sid == 0)
    def _zero_tail():
        pltpu.sync_copy(r0.at[pl.ds(0, TAIL)],
                        acc.at[pl.ds(NSUB * ROWS_PT, TAIL)])

    plsc.subcore_barrier()

    def compute(t, b):
        rw = rows[b]
        rd = rows_d[b]

        def group(g4, _):
            r16s = [(4 * g4 + k) * LANES + iota for k in range(4)]
            for hh in range(HEADS):
                ci = jnp.full((LANES,), ACOL + hh, jnp.int32)
                ch = jnp.full((LANES,), hh, jnp.int32)
                avs = [plsc.load_gather(rd, [r, ch]) for r in r16s]
                ass = [plsc.load_gather(rw, [r, ci]) for r in r16s]
                aas = [s + v for s, v in zip(ass, avs)]
                ws = [jnp.exp(jnp.maximum(a, 0.2 * a)) for a in aas]
                for r, w in zip(r16s, ws):
                    plsc.store_scatter(rw, [r, ci], w)
                for cc in range(CH):
                    col = jnp.full((LANES,), hh * CH + cc, jnp.int32)
                    vs = [plsc.load_gather(rw, [r, col]) for r in r16s]
                    for r, v, w in zip(r16s, vs, ws):
                        plsc.store_scatter(rw, [r, col], v * w)
            return 0

        lax.fori_loop(0, K // LANES // 4, group, 0)

    _edge_pipeline(
        wid, src2d_hbm, dst2d_hbm, src_all, dst_all,
        gathers=[(tab_hbm, rows, False), (tabd_hbm, rows_d, True)],
        scatters=[(acc, rows)],
        compute=compute, gsems=(gs0, gs1), ssems=(ss0, ss1))
    plsc.subcore_barrier()

    _over_my_rows(sid, lambda b, n: pltpu.sync_copy(
        acc.at[pl.ds(b, n)], out_hbm.at[cid, pl.ds(b, n)]))

    @pl.when(sid == 0)
    def _dump_tail():
        pltpu.sync_copy(acc.at[pl.ds(NSUB * ROWS_PT, TAIL)],
                        out_hbm.at[cid, pl.ds(NSUB * ROWS_PT, TAIL)])


def _sc2_body(tab2_hbm, tabd2_hbm, src2d_hbm, dst2d_hbm, out_hbm,
              acc, r0, r1, rd0, rd1, src_all, dst_all, gs0, gs1, ss0, ss1):
    cid = lax.axis_index("c")
    sid = lax.axis_index("s")
    wid = cid * NSUB + sid
    iota = _iota16()
    rows = (r0, r1)
    rows_d = (rd0, rd1)

    _zero_buf(r0, K, WT2)
    _over_my_rows(sid, lambda b, n: pltpu.sync_copy(
        r0.at[pl.ds(0, n)], acc.at[pl.ds(b, n)]))

    @pl.when(sid == 0)
    def _zero_tail():
        pltpu.sync_copy(r0.at[pl.ds(0, TAIL)],
                        acc.at[pl.ds(NSUB * ROWS_PT, TAIL)])

    plsc.subcore_barrier()

    def compute(t, b):
        rw = rows[b]
        rd = rows_d[b]

        def group(g4, _):
            r16s = [(4 * g4 + k) * LANES + iota for k in range(4)]
            cs = jnp.full((LANES,), NCLS + 1, jnp.int32)
            c0 = jnp.full((LANES,), 0, jnp.int32)
            avs = [plsc.load_gather(rd, [r, c0]) for r in r16s]
            ass = [plsc.load_gather(rw, [r, cs]) for r in r16s]
            aas = [s + v for s, v in zip(ass, avs)]
            ws = [jnp.exp(jnp.maximum(a, 0.2 * a)) for a in aas]
            for cc in range(CH):
                col = jnp.full((LANES,), cc, jnp.int32)
                vs = [plsc.load_gather(rw, [r, col]) for r in r16s]
                for r, v, w in zip(r16s, vs, ws):
                    plsc.store_scatter(rw, [r, col], v * w)
            return 0

        lax.fori_loop(0, K // LANES // 4, group, 0)

    _edge_pipeline(
        wid, src2d_hbm, dst2d_hbm, src_all, dst_all,
        gathers=[(tab2_hbm, rows, False), (tabd2_hbm, rows_d, True)],
        scatters=[(acc, rows)],
        compute=compute, gsems=(gs0, gs1), ssems=(ss0, ss1))
    plsc.subcore_barrier()

    _over_my_rows(sid, lambda b, n: pltpu.sync_copy(
        acc.at[pl.ds(b, n)], out_hbm.at[cid, pl.ds(b, n)]))

    @pl.when(sid == 0)
    def _dump_tail():
        pltpu.sync_copy(acc.at[pl.ds(NSUB * ROWS_PT, TAIL)],
                        out_hbm.at[cid, pl.ds(NSUB * ROWS_PT, TAIL)])


_sc1 = functools.partial(
    pl.kernel,
    out_type=jax.ShapeDtypeStruct((NCORE, N, ROW1), jnp.float32),
    mesh=_MESH,
    compiler_params=_SC_PARAMS,
    scratch_types=[
        pltpu.VMEM_SHARED((N, ROW1), jnp.float32),
        pltpu.VMEM((K, ROW1), jnp.float32),
        pltpu.VMEM((K, ROW1), jnp.float32),
        pltpu.VMEM((K, WA), jnp.float32),
        pltpu.VMEM((K, WA), jnp.float32),
        pltpu.VMEM((NT + 1, K), jnp.int32),
        pltpu.VMEM((NT + 1, K), jnp.int32),
        pltpu.SemaphoreType.DMA,
        pltpu.SemaphoreType.DMA,
        pltpu.SemaphoreType.DMA,
        pltpu.SemaphoreType.DMA,
    ],
)(_sc1_body)

_sc2 = functools.partial(
    pl.kernel,
    out_type=jax.ShapeDtypeStruct((NCORE, N, WT2), jnp.float32),
    mesh=_MESH,
    compiler_params=_SC_PARAMS,
    scratch_types=[
        pltpu.VMEM_SHARED((N, WT2), jnp.float32),
        pltpu.VMEM((K, WT2), jnp.float32),
        pltpu.VMEM((K, WT2), jnp.float32),
        pltpu.VMEM((K, HEADS), jnp.float32),
        pltpu.VMEM((K, HEADS), jnp.float32),
        pltpu.VMEM((NT + 1, K), jnp.int32),
        pltpu.VMEM((NT + 1, K), jnp.int32),
        pltpu.SemaphoreType.DMA,
        pltpu.SemaphoreType.DMA,
        pltpu.SemaphoreType.DMA,
        pltpu.SemaphoreType.DMA,
    ],
)(_sc2_body)


def kernel(x, edge_index, W1, att_src1, att_dst1, b1, W2, att_src2, att_dst2,
           b2):
    src = edge_index[0].reshape(NCHUNK, K)
    dst = edge_index[1].reshape(NCHUNK, K)

    eye = jnp.eye(HEADS, dtype=jnp.float32)
    As1 = (att_src1[:, :, None] * eye[:, None, :]).reshape(D1, HEADS)
    Ad1 = (att_dst1[:, :, None] * eye[:, None, :]).reshape(D1, HEADS)
    b64 = jnp.repeat(eye, CH, axis=1)
    pad8 = jnp.zeros((F_IN, WA - HEADS), jnp.float32)
    wtab = jnp.concatenate(
        [W1, W1 @ As1, jnp.zeros((F_IN, ROW1 - D1 - HEADS), jnp.float32)],
        axis=1)
    wad = jnp.concatenate([W1 @ Ad1, pad8], axis=1)

    tab1, tabd = _tc1(x, wtab, wad)
    p1 = _sc1(tab1, tabd, src, dst)
    tab2, tabd2 = _tc2(p1[0], p1[1], tab1, tabd, b1.reshape(1, D1),
                       W2, att_src2.T, att_dst2.T, b64)
    p2 = _sc2(tab2, tabd2, src, dst)
    return _tc3(p2[0], p2[1], tab2, tabd2, b2.reshape(1, NCLS))
